# Initial kernel scaffold; baseline (speedup 1.0000x reference)
import jax
import jax.numpy as jnp
from jax.experimental import pallas as pl


def kernel(user_idx, pos_item, neg_item, edge_index, ig_edge_index, kg_edge_index, cf_edge_index, emb, i_Wt, i_bt, i_W1, i_b1, i_W2, i_b2, c_Wt, c_bt, c_W1, c_b1, c_W2, c_b2):
    raise NotImplementedError("write your pallas kernel here")


# speedup vs baseline: 13.6476x; 13.6476x over previous
import functools
import math

import jax
import jax.numpy as jnp
from jax import lax
from jax.experimental import pallas as pl
from jax.experimental.pallas import tpu as pltpu
from jax.experimental.pallas import tpu_sc as plsc

N = 10000
EMB = 128
E = 320000
NGRAPH = 4
LAYERS = 2
STEPS = 5
NOISE_SCALE = 0.1
NOISE_MIN = 0.0001
NOISE_MAX = 0.02
CF_WEIGHT = 1.0
WEIGHT_DECAY = 0.0001
GCL_WEIGHT = 0.1
GCL_TEMP = 0.2
D_EMB = 10
B = 4096

NC = 2
NS = 16
CH = 128
NBUF = 4
NBUF_P = 2
CHUNKS = 160
EPT = CHUNKS * CH
EPAD = NS * EPT
NP = 10240
RPT = NP // NS

RB = 1000
NB = N // RB


_sc_mesh = plsc.VectorSubcoreMesh(core_axis_name="c", subcore_axis_name="s")


@functools.partial(
    pl.kernel,
    out_type=jax.ShapeDtypeStruct((NGRAPH * NP,), jnp.float32),
    mesh=_sc_mesh,
    scratch_types=[
        pltpu.VMEM_SHARED((NP,), jnp.float32),
        pltpu.VMEM_SHARED((NP,), jnp.float32),
        pltpu.VMEM((CH,), jnp.float32),
        *[pltpu.VMEM((CH,), jnp.int32) for _ in range(NBUF)],
        *[pltpu.SemaphoreType.DMA for _ in range(NBUF)],
    ],
)
def _sc_degree(dsts, zeros1, deg_out, acc0, acc1, ones_v, *rest):
    idx = rest[:NBUF]
    sem = rest[NBUF:]
    cid = lax.axis_index("c")
    sid = lax.axis_index("s")
    accs = [acc0, acc1]
    for j in range(CH // 16):
        ones_v[pl.ds(j * 16, 16)] = jnp.ones((16,), jnp.float32)
    for gl in range(2):
        pltpu.sync_copy(zeros1.at[pl.ds(sid * RPT, RPT)],
                        accs[gl].at[pl.ds(sid * RPT, RPT)])
    plsc.subcore_barrier()
    for gl in range(2):
        g = cid * 2 + gl
        ebase = g * EPAD + sid * EPT

        def body(it, _, gl=gl, ebase=ebase):
            cps = []
            for b in range(NBUF):
                off = pl.multiple_of(ebase + (it * NBUF + b) * CH, CH)
                cps.append(pltpu.async_copy(
                    dsts.at[pl.ds(off, CH)], idx[b], sem[b]))
            for b in range(NBUF):
                cps[b].wait()
                pltpu.sync_copy(ones_v, accs[gl].at[idx[b]], add=True)
            return 0

        lax.fori_loop(0, CHUNKS // NBUF, body, 0)
    plsc.subcore_barrier()
    for gl in range(2):
        g = cid * 2 + gl
        off = pl.multiple_of(g * NP + sid * RPT, 8)
        pltpu.sync_copy(accs[gl].at[pl.ds(sid * RPT, RPT)],
                        deg_out.at[pl.ds(off, RPT)])


@functools.partial(
    pl.kernel,
    out_type=jax.ShapeDtypeStruct((NGRAPH * NP, EMB), jnp.float32),
    mesh=_sc_mesh,
    scratch_types=[
        pltpu.VMEM_SHARED((NP, EMB), jnp.float32),
        *[pltpu.VMEM((CH,), jnp.int32) for _ in range(NBUF_P)],
        *[pltpu.VMEM((CH,), jnp.int32) for _ in range(NBUF_P)],
        *[pltpu.VMEM((CH, EMB), jnp.float32) for _ in range(NBUF_P)],
        *[pltpu.SemaphoreType.DMA for _ in range(3 * NBUF_P)],
    ],
)
def _sc_propagate(srcs, dsts, table, zeros2, out, acc, *rest):
    sidx = rest[:NBUF_P]
    didx = rest[NBUF_P:2 * NBUF_P]
    rows = rest[2 * NBUF_P:3 * NBUF_P]
    sem_s = rest[3 * NBUF_P:4 * NBUF_P]
    sem_d = rest[4 * NBUF_P:5 * NBUF_P]
    sem_g = rest[5 * NBUF_P:6 * NBUF_P]
    cid = lax.axis_index("c")
    sid = lax.axis_index("s")
    for gl in range(2):
        g = cid * 2 + gl
        pltpu.sync_copy(zeros2.at[pl.ds(sid * RPT, RPT)],
                        acc.at[pl.ds(sid * RPT, RPT)])
        plsc.subcore_barrier()
        ebase = g * EPAD + sid * EPT

        def body(it, _, ebase=ebase):
            scp, dcp, gcp = [], [], []
            for b in range(NBUF_P):
                off = pl.multiple_of(ebase + (it * NBUF_P + b) * CH, CH)
                scp.append(pltpu.async_copy(
                    srcs.at[pl.ds(off, CH)], sidx[b], sem_s[b]))
                dcp.append(pltpu.async_copy(
                    dsts.at[pl.ds(off, CH)], didx[b], sem_d[b]))
            for b in range(NBUF_P):
                scp[b].wait()
                gcp.append(pltpu.async_copy(
                    table.at[sidx[b]], rows[b], sem_g[b]))
            for b in range(NBUF_P):
                gcp[b].wait()
                dcp[b].wait()
                pltpu.sync_copy(rows[b], acc.at[didx[b]], add=True)
            return 0

        lax.fori_loop(0, CHUNKS // NBUF_P, body, 0)
        plsc.subcore_barrier()
        off = g * NP + sid * RPT
        pltpu.sync_copy(acc.at[pl.ds(sid * RPT, RPT)],
                        out.at[pl.ds(off, RPT)])
        plsc.subcore_barrier()


NIDX = 6 * B
GPW = NIDX // (NC * NS)
GCH = GPW // CH


@functools.partial(
    pl.kernel,
    out_type=jax.ShapeDtypeStruct((NIDX, EMB), jnp.float32),
    mesh=_sc_mesh,
    scratch_types=[
        *[pltpu.VMEM((CH,), jnp.int32) for _ in range(2)],
        *[pltpu.VMEM((CH, EMB), jnp.float32) for _ in range(2)],
        *[pltpu.SemaphoreType.DMA for _ in range(4)],
    ],
)
def _sc_gather_rows(idx_all, tables, out, i0, i1, r0, r1, si0, si1, sg0, sg1):
    cid = lax.axis_index("c")
    sid = lax.axis_index("s")
    wid = sid * NC + cid
    base = wid * GPW
    idx = [i0, i1]
    rows = [r0, r1]
    sem_i = [si0, si1]
    sem_g = [sg0, sg1]
    for k in range(GCH):
        b = k % 2
        off = pl.multiple_of(base + k * CH, CH)
        pltpu.async_copy(idx_all.at[pl.ds(off, CH)], idx[b], sem_i[b]).wait()
        pltpu.async_copy(tables.at[idx[b]], rows[b], sem_g[b]).wait()
        pltpu.sync_copy(rows[b], out.at[pl.ds(off, CH)])


def _dinv_of(degcol):
    return jnp.where(degcol > 0.0, lax.rsqrt(jnp.maximum(degcol, 1e-30)), 0.0)


def _scale_table_kernel(degp_ref, x_ref, out_ref, *, power):
    g = pl.program_id(0)
    degs = degp_ref[...]
    col = jnp.zeros_like(degs[:, 0:1])
    for k in range(NGRAPH):
        col = col + jnp.where(g == k, degs[:, k:k + 1], 0.0)
    d = _dinv_of(col)
    w = d * d if power == 2 else d
    out_ref[...] = w * x_ref[...]


def _scale_emb(degp, embp):
    out = pl.pallas_call(
        functools.partial(_scale_table_kernel, power=1),
        grid=(NGRAPH, NP // 1024),
        in_specs=[
            pl.BlockSpec((1024, NGRAPH), lambda g, i: (i, 0)),
            pl.BlockSpec((1024, EMB), lambda g, i: (i, 0)),
        ],
        out_specs=pl.BlockSpec((1024, EMB), lambda g, i: (g * (NP // 1024) + i, 0)),
        out_shape=jax.ShapeDtypeStruct((NGRAPH * NP, EMB), jnp.float32),
    )(degp, embp)
    return out


def _scale_s1(degp, s1):
    out = pl.pallas_call(
        functools.partial(_scale_table_kernel, power=2),
        grid=(NGRAPH, NP // 1024),
        in_specs=[
            pl.BlockSpec((1024, NGRAPH), lambda g, i: (i, 0)),
            pl.BlockSpec((1024, EMB), lambda g, i: (g * (NP // 1024) + i, 0)),
        ],
        out_specs=pl.BlockSpec((1024, EMB), lambda g, i: (g * (NP // 1024) + i, 0)),
        out_shape=jax.ShapeDtypeStruct((NGRAPH * NP, EMB), jnp.float32),
    )(degp, s1)
    return out


def _rownorm(x):
    n = jnp.sqrt(jnp.sum(x * x, axis=-1, keepdims=True))
    return x / jnp.maximum(n, 1e-12)


def _main_kernel(emb_ref, s1_0, s1_1, s1_2, s1_3, s2_0, s2_1, s2_2, s2_3,
                 degp_ref, n1_ref, n2_ref, sab_ref, sbb_ref, te_ref,
                 iWt, ibt, iW1a, iW1b, ib1, iW2, ib2,
                 cWt, cbt, cW1a, cW1b, cb1, cW2, cb2,
                 all_ref, l4_ref, l2n_ref, l3n_ref, acc_ref):
    i = pl.program_id(0)
    deg = degp_ref[...]
    emb = emb_ref[...]

    def allg(g, s1b, s2b):
        d = _dinv_of(deg[:, g:g + 1])
        return (emb + d * (s1b[...] + s2b[...])) * (1.0 / 3.0)

    all_layer = allg(0, s1_0, s2_0)
    l2 = allg(1, s1_1, s2_1)
    l3 = allg(2, s1_2, s2_2)
    l1n = _rownorm(allg(3, s1_3, s2_3))

    f32 = jnp.float32

    def mlp(x_t, Wt, bt, W1a, W1b, b1, W2, b2):
        temb = jnp.dot(te_ref[...], Wt[...], preferred_element_type=f32) + bt[...]
        h = jnp.tanh(jnp.dot(x_t, W1a[...], preferred_element_type=f32)
                     + jnp.dot(temb, W1b[...], preferred_element_type=f32)
                     + b1[...])
        return jnp.dot(h, W2[...], preferred_element_type=f32) + b2[...]

    sab = sab_ref[...]
    sbb = sbb_ref[...]
    x_t2 = sab * l2 + sbb * n1_ref[...]
    x_t3 = sab * l3 + sbb * n2_ref[...]
    d2 = mlp(x_t2, iWt, ibt, iW1a, iW1b, ib1, iW2, ib2)
    d3 = mlp(x_t3, cWt, cbt, cW1a, cW1b, cb1, cW2, cb2)

    part = (jnp.sum((l1n - d2) ** 2) + jnp.sum((l1n - d3) ** 2)) * (1.0 / EMB)

    all_ref[...] = all_layer
    l4_ref[...] = _rownorm(all_layer)
    l2n_ref[...] = _rownorm(l2 + d2)
    l3n_ref[...] = _rownorm(l3 + d3)

    @pl.when(i == 0)
    def _():
        acc_ref[...] = jnp.zeros((1, 1), f32)

    acc_ref[...] = acc_ref[...] + jnp.reshape(part, (1, 1))


def _run_main(embk, s1r, s2r, degp, n1, n2, sab, sbb, te, wts):
    rowspec = pl.BlockSpec((RB, EMB), lambda i: (i, 0))
    wspecs = []
    for w in wts:
        wspecs.append(pl.BlockSpec(w.shape, lambda i: (0,) * w.ndim))
    outs = pl.pallas_call(
        _main_kernel,
        grid=(NB,),
        in_specs=[
            rowspec,
            *[rowspec] * 8,
            pl.BlockSpec((RB, NGRAPH), lambda i: (i, 0)),
            rowspec, rowspec,
            pl.BlockSpec((RB, 1), lambda i: (i, 0)),
            pl.BlockSpec((RB, 1), lambda i: (i, 0)),
            pl.BlockSpec((RB, D_EMB), lambda i: (i, 0)),
            *wspecs,
        ],
        out_specs=[
            rowspec, rowspec, rowspec, rowspec,
            pl.BlockSpec((1, 1), lambda i: (0, 0)),
        ],
        out_shape=[
            jax.ShapeDtypeStruct((N, EMB), jnp.float32),
            jax.ShapeDtypeStruct((N, EMB), jnp.float32),
            jax.ShapeDtypeStruct((N, EMB), jnp.float32),
            jax.ShapeDtypeStruct((N, EMB), jnp.float32),
            jax.ShapeDtypeStruct((1, 1), jnp.float32),
        ],
        compiler_params=pltpu.CompilerParams(
            dimension_semantics=("arbitrary",)),
    )(embk, s1r[0], s1r[1], s1r[2], s1r[3], s2r[0], s2r[1], s2r[2], s2r[3],
      degp, n1, n2, sab, sbb, te, *wts)
    return outs


def _infonce_kernel(l4_ref, b2_ref, b3_ref, acc_ref,
                    m2, s2, p2, m3, s3, p3):
    i = pl.program_id(0)
    j = pl.program_id(1)
    a = l4_ref[...] * (1.0 / GCL_TEMP)
    f32 = jnp.float32
    dn = (((1,), (1,)), ((), ()))
    log2 = lax.dot_general(a, b2_ref[...], dn, preferred_element_type=f32)
    log3 = lax.dot_general(a, b3_ref[...], dn, preferred_element_type=f32)

    rid = lax.broadcasted_iota(jnp.int32, (RB, RB), 0)
    cid = lax.broadcasted_iota(jnp.int32, (RB, RB), 1)
    is_diag = rid == cid

    def update(lg, m_ref, s_ref, p_ref):
        mj = jnp.max(lg, axis=1, keepdims=True)

        @pl.when(j == 0)
        def _():
            m_ref[...] = mj
            s_ref[...] = jnp.sum(jnp.exp(lg - mj), axis=1, keepdims=True)

        @pl.when(j > 0)
        def _():
            mold = m_ref[...]
            mnew = jnp.maximum(mold, mj)
            s_ref[...] = (s_ref[...] * jnp.exp(mold - mnew)
                          + jnp.sum(jnp.exp(lg - mnew), axis=1, keepdims=True))
            m_ref[...] = mnew

        @pl.when(j == i)
        def _():
            p_ref[...] = jnp.sum(jnp.where(is_diag, lg, 0.0), axis=1,
                                 keepdims=True)

    update(log2, m2, s2, p2)
    update(log3, m3, s3, p3)

    @pl.when((i == 0) & (j == 0))
    def _():
        acc_ref[...] = jnp.zeros((1, 1), f32)

    @pl.when(j == NB - 1)
    def _():
        part = (jnp.sum(m2[...] + jnp.log(s2[...]) - p2[...])
                + jnp.sum(m3[...] + jnp.log(s3[...]) - p3[...]))
        acc_ref[...] = acc_ref[...] + jnp.reshape(part, (1, 1))


def _run_infonce(l4, l2n, l3n):
    col = pl.BlockSpec((RB, 1), None)
    acc = pl.pallas_call(
        _infonce_kernel,
        grid=(NB, NB),
        in_specs=[
            pl.BlockSpec((RB, EMB), lambda i, j: (i, 0)),
            pl.BlockSpec((RB, EMB), lambda i, j: (j, 0)),
            pl.BlockSpec((RB, EMB), lambda i, j: (j, 0)),
        ],
        out_specs=pl.BlockSpec((1, 1), lambda i, j: (0, 0)),
        out_shape=jax.ShapeDtypeStruct((1, 1), jnp.float32),
        scratch_shapes=[pltpu.VMEM((RB, 1), jnp.float32) for _ in range(6)],
        compiler_params=pltpu.CompilerParams(
            dimension_semantics=("arbitrary", "arbitrary")),
    )(l4, l2n, l3n)
    return acc


def _bpr_kernel(g_ref, out_ref):
    u = g_ref[0]
    p = g_ref[1]
    ng = g_ref[2]
    ue = g_ref[3]
    pe = g_ref[4]
    ne = g_ref[5]
    pos = jnp.sum(u * p, axis=-1)
    neg = jnp.sum(u * ng, axis=-1)
    x = pos - neg
    logsig = jnp.minimum(x, 0.0) - jnp.log1p(jnp.exp(-jnp.abs(x)))
    cf = -jnp.sum(logsig) * (1.0 / B)
    reg = 0.5 * (jnp.sum(ue * ue) + jnp.sum(pe * pe) + jnp.sum(ne * ne)) / B
    out_ref[...] = jnp.reshape(CF_WEIGHT * cf + WEIGHT_DECAY * reg, (1, 1))


def _run_bpr(grows):
    return pl.pallas_call(
        _bpr_kernel,
        in_specs=[pl.BlockSpec((6, B, EMB), lambda: (0, 0, 0))],
        out_specs=pl.BlockSpec((1, 1), lambda: (0, 0)),
        out_shape=jax.ShapeDtypeStruct((1, 1), jnp.float32),
    )(grows)


def _timestep_embedding_const():
    ts = jnp.arange(N, dtype=jnp.float32) % STEPS
    half = D_EMB // 2
    freqs = jnp.exp(-math.log(10000.0)
                    * jnp.arange(half, dtype=jnp.float32) / half)
    a = ts[:, None] * freqs[None, :]
    return jnp.concatenate([jnp.cos(a), jnp.sin(a)], axis=-1)


def kernel(user_idx, pos_item, neg_item, edge_index, ig_edge_index,
           kg_edge_index, cf_edge_index, emb, i_Wt, i_bt, i_W1, i_b1, i_W2,
           i_b2, c_Wt, c_bt, c_W1, c_b1, c_W2, c_b2):
    i32 = jnp.int32
    f32 = jnp.float32

    eis = [edge_index, ig_edge_index, kg_edge_index, cf_edge_index]
    pad_n = EPAD - E
    pad_src = (jnp.arange(pad_n, dtype=i32) * 37) % N
    pad_dst = N + (jnp.arange(pad_n, dtype=i32) % (NP - N))
    srcs = jnp.stack([ei[0].astype(i32) for ei in eis])
    dsts = jnp.stack([ei[1].astype(i32) for ei in eis])
    srcs = jnp.concatenate(
        [srcs, jnp.broadcast_to(pad_src, (NGRAPH, pad_n))], axis=1)
    dsts = jnp.concatenate(
        [dsts, jnp.broadcast_to(pad_dst, (NGRAPH, pad_n))], axis=1)
    goff = (jnp.arange(NGRAPH, dtype=i32) * NP)[:, None]
    srcs_shift = (srcs + goff).reshape(-1)
    dsts_flat = dsts.reshape(-1)

    zeros1 = jnp.zeros((NP,), f32)
    zeros2 = jnp.zeros((NP, EMB), f32)

    deg_flat = _sc_degree(dsts_flat, zeros1)
    degp = deg_flat.reshape(NGRAPH, NP).T

    embp = jnp.concatenate([emb.astype(f32),
                            jnp.zeros((NP - N, EMB), f32)], axis=0)
    y1 = _scale_emb(degp, embp)
    s1 = _sc_propagate(srcs_shift, dsts_flat, y1, zeros2)
    y2 = _scale_s1(degp, s1)
    s2 = _sc_propagate(srcs_shift, dsts_flat, y2, zeros2)

    s1r = s1.reshape(NGRAPH, NP, EMB)
    s2r = s2.reshape(NGRAPH, NP, EMB)
    s1g = [s1r[g] for g in range(NGRAPH)]
    s2g = [s2r[g] for g in range(NGRAPH)]

    betas = NOISE_SCALE * jnp.linspace(NOISE_MIN, NOISE_MAX, STEPS)
    ab = jnp.cumprod(1.0 - betas)
    ts = jnp.arange(N) % STEPS
    abt = ab[ts][:, None].astype(f32)
    sab = jnp.sqrt(abt)
    sbb = jnp.sqrt(1.0 - abt)
    n1 = jax.random.normal(jax.random.key(1), (N, EMB), dtype=f32)
    n2 = jax.random.normal(jax.random.key(2), (N, EMB), dtype=f32)
    te = _timestep_embedding_const()

    wts = [i_Wt, i_bt.reshape(1, D_EMB), i_W1[:EMB], i_W1[EMB:],
           i_b1.reshape(1, EMB), i_W2, i_b2.reshape(1, EMB),
           c_Wt, c_bt.reshape(1, D_EMB), c_W1[:EMB], c_W1[EMB:],
           c_b1.reshape(1, EMB), c_W2, c_b2.reshape(1, EMB)]

    all_layer, l4, l2n, l3n, diff_acc = _run_main(
        emb.astype(f32), s1g, s2g, degp[:N], n1, n2, sab, sbb, te, wts)

    nce_acc = _run_infonce(l4, l2n, l3n)

    tables = jnp.concatenate([all_layer, emb.astype(f32)], axis=0)
    idx_all = jnp.concatenate([
        user_idx.astype(i32), pos_item.astype(i32), neg_item.astype(i32),
        user_idx.astype(i32) + N, pos_item.astype(i32) + N,
        neg_item.astype(i32) + N])
    grows = _sc_gather_rows(idx_all, tables).reshape(6, B, EMB)
    bpr = _run_bpr(grows)

    diff_loss = diff_acc[0, 0] * (1.0 / N)
    gcl = nce_acc[0, 0] * (1.0 / N)
    return bpr[0, 0] + diff_loss + GCL_WEIGHT * gcl


# speedup vs baseline: 15.0648x; 1.1038x over previous
import functools
import math

import jax
import jax.numpy as jnp
from jax import lax
from jax.experimental import pallas as pl
from jax.experimental.pallas import tpu as pltpu
from jax.experimental.pallas import tpu_sc as plsc

N = 10000
EMB = 128
E = 320000
NGRAPH = 4
LAYERS = 2
STEPS = 5
NOISE_SCALE = 0.1
NOISE_MIN = 0.0001
NOISE_MAX = 0.02
CF_WEIGHT = 1.0
WEIGHT_DECAY = 0.0001
GCL_WEIGHT = 0.1
GCL_TEMP = 0.2
D_EMB = 10
B = 4096

NC = 2
NS = 16
CH = 128
NBUF = 4
NBUF_P = 2
CHUNKS = 160
EPT = CHUNKS * CH
EPAD = NS * EPT
NP = 10240
RPT = NP // NS

RB = 1000
NB = N // RB


_sc_mesh = plsc.VectorSubcoreMesh(core_axis_name="c", subcore_axis_name="s")


@functools.partial(
    pl.kernel,
    out_type=jax.ShapeDtypeStruct((NGRAPH * NP,), jnp.float32),
    mesh=_sc_mesh,
    scratch_types=[
        pltpu.VMEM_SHARED((NP,), jnp.float32),
        pltpu.VMEM_SHARED((NP,), jnp.float32),
        pltpu.VMEM((CH,), jnp.float32),
        *[pltpu.VMEM((CH,), jnp.int32) for _ in range(NBUF)],
        *[pltpu.SemaphoreType.DMA for _ in range(NBUF)],
    ],
)
def _sc_degree(dsts, zeros1, deg_out, acc0, acc1, ones_v, *rest):
    idx = rest[:NBUF]
    sem = rest[NBUF:]
    cid = lax.axis_index("c")
    sid = lax.axis_index("s")
    accs = [acc0, acc1]
    for j in range(CH // 16):
        ones_v[pl.ds(j * 16, 16)] = jnp.ones((16,), jnp.float32)
    for gl in range(2):
        pltpu.sync_copy(zeros1.at[pl.ds(sid * RPT, RPT)],
                        accs[gl].at[pl.ds(sid * RPT, RPT)])
    plsc.subcore_barrier()
    for gl in range(2):
        g = cid * 2 + gl
        ebase = g * EPAD + sid * EPT

        def body(it, _, gl=gl, ebase=ebase):
            cps = []
            for b in range(NBUF):
                off = pl.multiple_of(ebase + (it * NBUF + b) * CH, CH)
                cps.append(pltpu.async_copy(
                    dsts.at[pl.ds(off, CH)], idx[b], sem[b]))
            for b in range(NBUF):
                cps[b].wait()
                pltpu.sync_copy(ones_v, accs[gl].at[idx[b]], add=True)
            return 0

        lax.fori_loop(0, CHUNKS // NBUF, body, 0)
    plsc.subcore_barrier()
    for gl in range(2):
        g = cid * 2 + gl
        off = pl.multiple_of(g * NP + sid * RPT, 8)
        pltpu.sync_copy(accs[gl].at[pl.ds(sid * RPT, RPT)],
                        deg_out.at[pl.ds(off, RPT)])


@functools.partial(
    pl.kernel,
    out_type=jax.ShapeDtypeStruct((NGRAPH * NP, EMB), jnp.float32),
    mesh=_sc_mesh,
    scratch_types=[
        pltpu.VMEM_SHARED((NP, EMB), jnp.float32),
        pltpu.VMEM((8, CH), jnp.int32),
        pltpu.VMEM((8, CH), jnp.int32),
        *[pltpu.VMEM((CH, EMB), jnp.float32) for _ in range(2)],
        *[pltpu.SemaphoreType.DMA for _ in range(6)],
    ],
)
def _sc_propagate(idxcat, table, zeros2, out, acc, bx, by, r0, r1,
                  smx, smy, sg0, sg1, ss0, ss1):
    cid = lax.axis_index("c")
    sid = lax.axis_index("s")
    rows = [r0, r1]
    sem_g = [sg0, sg1]
    sem_s = [ss0, ss1]
    nbody = CHUNKS // 8

    def fire_idx(buf, sem, g, body_ix, half):
        base = (g * NS + sid) * (2 * CHUNKS)
        off = base + body_ix * 16 + half * 8
        return pltpu.async_copy(idxcat.at[pl.ds(off, 8)], buf, sem)

    def fire_g(ib, j, b):
        return pltpu.async_copy(table.at[ib.at[2 * j]], rows[b], sem_g[b])

    def fire_s(ib, j, b):
        return pltpu.async_copy(rows[b], acc.at[ib.at[2 * j + 1]],
                                sem_s[b], add=True)

    def wait(sem, ref):
        if ref is bx or ref is by:
            dummy = idxcat.at[pl.ds(0, 8)]
        else:
            dummy = table.at[pl.ds(0, CH)]
        pltpu.make_async_copy(dummy, ref, sem).wait()

    for gl in range(2):
        g = cid * 2 + gl
        pltpu.sync_copy(zeros2.at[pl.ds(sid * RPT, RPT)],
                        acc.at[pl.ds(sid * RPT, RPT)])
        plsc.subcore_barrier()

        fire_idx(bx, smx, g, 0, 0).wait()
        fire_idx(by, smy, g, 0, 1)
        fire_g(bx, 0, 0)
        fire_g(bx, 1, 1)

        def body(k, _, g=g):
            last = k == nbody - 1
            wait(sem_g[0], rows[0]); fire_s(bx, 0, 0)
            wait(sem_g[1], rows[1]); fire_s(bx, 1, 1)
            wait(sem_s[0], rows[0]); fire_g(bx, 2, 0)
            wait(sem_s[1], rows[1]); fire_g(bx, 3, 1)
            wait(smy, by)
            wait(sem_g[0], rows[0]); fire_s(bx, 2, 0)
            wait(sem_g[1], rows[1]); fire_s(bx, 3, 1)
            wait(sem_s[0], rows[0]); fire_g(by, 0, 0)
            wait(sem_s[1], rows[1]); fire_g(by, 1, 1)

            @pl.when(jnp.logical_not(last))
            def _():
                fire_idx(bx, smx, g, k + 1, 0)

            wait(sem_g[0], rows[0]); fire_s(by, 0, 0)
            wait(sem_g[1], rows[1]); fire_s(by, 1, 1)
            wait(sem_s[0], rows[0]); fire_g(by, 2, 0)
            wait(sem_s[1], rows[1]); fire_g(by, 3, 1)
            wait(sem_g[0], rows[0]); fire_s(by, 2, 0)
            wait(sem_g[1], rows[1]); fire_s(by, 3, 1)
            wait(sem_s[0], rows[0])

            @pl.when(jnp.logical_not(last))
            def _():
                wait(smx, bx)
                fire_g(bx, 0, 0)

            wait(sem_s[1], rows[1])

            @pl.when(jnp.logical_not(last))
            def _():
                fire_g(bx, 1, 1)
                fire_idx(by, smy, g, k + 1, 1)

            return 0

        lax.fori_loop(0, nbody, body, 0)
        plsc.subcore_barrier()
        off = g * NP + sid * RPT
        pltpu.sync_copy(acc.at[pl.ds(sid * RPT, RPT)],
                        out.at[pl.ds(off, RPT)])
        plsc.subcore_barrier()


NIDX = 6 * B
GPW = NIDX // (NC * NS)
GCH = GPW // CH


@functools.partial(
    pl.kernel,
    out_type=jax.ShapeDtypeStruct((NIDX, EMB), jnp.float32),
    mesh=_sc_mesh,
    scratch_types=[
        *[pltpu.VMEM((CH,), jnp.int32) for _ in range(2)],
        *[pltpu.VMEM((CH, EMB), jnp.float32) for _ in range(2)],
        *[pltpu.SemaphoreType.DMA for _ in range(4)],
    ],
)
def _sc_gather_rows(idx_all, tables, out, i0, i1, r0, r1, si0, si1, sg0, sg1):
    cid = lax.axis_index("c")
    sid = lax.axis_index("s")
    wid = sid * NC + cid
    base = wid * GPW
    idx = [i0, i1]
    rows = [r0, r1]
    sem_i = [si0, si1]
    sem_g = [sg0, sg1]
    for k in range(GCH):
        b = k % 2
        off = pl.multiple_of(base + k * CH, CH)
        pltpu.async_copy(idx_all.at[pl.ds(off, CH)], idx[b], sem_i[b]).wait()
        pltpu.async_copy(tables.at[idx[b]], rows[b], sem_g[b]).wait()
        pltpu.sync_copy(rows[b], out.at[pl.ds(off, CH)])


def _dinv_of(degcol):
    return jnp.where(degcol > 0.0, lax.rsqrt(jnp.maximum(degcol, 1e-30)), 0.0)


def _scale_table_kernel(degp_ref, x_ref, out_ref, *, power):
    g = pl.program_id(0)
    degs = degp_ref[...]
    col = jnp.zeros_like(degs[:, 0:1])
    for k in range(NGRAPH):
        col = col + jnp.where(g == k, degs[:, k:k + 1], 0.0)
    d = _dinv_of(col)
    w = d * d if power == 2 else d
    out_ref[...] = w * x_ref[...]


def _scale_emb(degp, embp):
    out = pl.pallas_call(
        functools.partial(_scale_table_kernel, power=1),
        grid=(NGRAPH, NP // 1024),
        in_specs=[
            pl.BlockSpec((1024, NGRAPH), lambda g, i: (i, 0)),
            pl.BlockSpec((1024, EMB), lambda g, i: (i, 0)),
        ],
        out_specs=pl.BlockSpec((1024, EMB), lambda g, i: (g * (NP // 1024) + i, 0)),
        out_shape=jax.ShapeDtypeStruct((NGRAPH * NP, EMB), jnp.float32),
    )(degp, embp)
    return out


def _scale_s1(degp, s1):
    out = pl.pallas_call(
        functools.partial(_scale_table_kernel, power=2),
        grid=(NGRAPH, NP // 1024),
        in_specs=[
            pl.BlockSpec((1024, NGRAPH), lambda g, i: (i, 0)),
            pl.BlockSpec((1024, EMB), lambda g, i: (g * (NP // 1024) + i, 0)),
        ],
        out_specs=pl.BlockSpec((1024, EMB), lambda g, i: (g * (NP // 1024) + i, 0)),
        out_shape=jax.ShapeDtypeStruct((NGRAPH * NP, EMB), jnp.float32),
    )(degp, s1)
    return out


def _rownorm(x):
    n = jnp.sqrt(jnp.sum(x * x, axis=-1, keepdims=True))
    return x / jnp.maximum(n, 1e-12)


def _main_kernel(emb_ref, s1_0, s1_1, s1_2, s1_3, s2_0, s2_1, s2_2, s2_3,
                 degp_ref, n1_ref, n2_ref, sab_ref, sbb_ref, te_ref,
                 iWt, ibt, iW1a, iW1b, ib1, iW2, ib2,
                 cWt, cbt, cW1a, cW1b, cb1, cW2, cb2,
                 all_ref, l4_ref, l2n_ref, l3n_ref, acc_ref):
    i = pl.program_id(0)
    deg = degp_ref[...]
    emb = emb_ref[...]

    def allg(g, s1b, s2b):
        d = _dinv_of(deg[:, g:g + 1])
        return (emb + d * (s1b[...] + s2b[...])) * (1.0 / 3.0)

    all_layer = allg(0, s1_0, s2_0)
    l2 = allg(1, s1_1, s2_1)
    l3 = allg(2, s1_2, s2_2)
    l1n = _rownorm(allg(3, s1_3, s2_3))

    f32 = jnp.float32

    def mlp(x_t, Wt, bt, W1a, W1b, b1, W2, b2):
        temb = jnp.dot(te_ref[...], Wt[...], preferred_element_type=f32) + bt[...]
        h = jnp.tanh(jnp.dot(x_t, W1a[...], preferred_element_type=f32)
                     + jnp.dot(temb, W1b[...], preferred_element_type=f32)
                     + b1[...])
        return jnp.dot(h, W2[...], preferred_element_type=f32) + b2[...]

    sab = sab_ref[...]
    sbb = sbb_ref[...]
    x_t2 = sab * l2 + sbb * n1_ref[...]
    x_t3 = sab * l3 + sbb * n2_ref[...]
    d2 = mlp(x_t2, iWt, ibt, iW1a, iW1b, ib1, iW2, ib2)
    d3 = mlp(x_t3, cWt, cbt, cW1a, cW1b, cb1, cW2, cb2)

    part = (jnp.sum((l1n - d2) ** 2) + jnp.sum((l1n - d3) ** 2)) * (1.0 / EMB)

    all_ref[...] = all_layer
    l4_ref[...] = _rownorm(all_layer)
    l2n_ref[...] = _rownorm(l2 + d2)
    l3n_ref[...] = _rownorm(l3 + d3)

    @pl.when(i == 0)
    def _():
        acc_ref[...] = jnp.zeros((1, 1), f32)

    acc_ref[...] = acc_ref[...] + jnp.reshape(part, (1, 1))


def _run_main(embk, s1r, s2r, degp, n1, n2, sab, sbb, te, wts):
    rowspec = pl.BlockSpec((RB, EMB), lambda i: (i, 0))
    wspecs = []
    for w in wts:
        wspecs.append(pl.BlockSpec(w.shape, lambda i: (0,) * w.ndim))
    outs = pl.pallas_call(
        _main_kernel,
        grid=(NB,),
        in_specs=[
            rowspec,
            *[rowspec] * 8,
            pl.BlockSpec((RB, NGRAPH), lambda i: (i, 0)),
            rowspec, rowspec,
            pl.BlockSpec((RB, 1), lambda i: (i, 0)),
            pl.BlockSpec((RB, 1), lambda i: (i, 0)),
            pl.BlockSpec((RB, D_EMB), lambda i: (i, 0)),
            *wspecs,
        ],
        out_specs=[
            rowspec, rowspec, rowspec, rowspec,
            pl.BlockSpec((1, 1), lambda i: (0, 0)),
        ],
        out_shape=[
            jax.ShapeDtypeStruct((N, EMB), jnp.float32),
            jax.ShapeDtypeStruct((N, EMB), jnp.float32),
            jax.ShapeDtypeStruct((N, EMB), jnp.float32),
            jax.ShapeDtypeStruct((N, EMB), jnp.float32),
            jax.ShapeDtypeStruct((1, 1), jnp.float32),
        ],
        compiler_params=pltpu.CompilerParams(
            dimension_semantics=("arbitrary",)),
    )(embk, s1r[0], s1r[1], s1r[2], s1r[3], s2r[0], s2r[1], s2r[2], s2r[3],
      degp, n1, n2, sab, sbb, te, *wts)
    return outs


def _infonce_kernel(l4_ref, b2_ref, b3_ref, acc_ref,
                    m2, s2, p2, m3, s3, p3):
    i = pl.program_id(0)
    j = pl.program_id(1)
    a = l4_ref[...] * (1.0 / GCL_TEMP)
    f32 = jnp.float32
    dn = (((1,), (1,)), ((), ()))
    log2 = lax.dot_general(a, b2_ref[...], dn, preferred_element_type=f32)
    log3 = lax.dot_general(a, b3_ref[...], dn, preferred_element_type=f32)

    rid = lax.broadcasted_iota(jnp.int32, (RB, RB), 0)
    cid = lax.broadcasted_iota(jnp.int32, (RB, RB), 1)
    is_diag = rid == cid

    def update(lg, m_ref, s_ref, p_ref):
        mj = jnp.max(lg, axis=1, keepdims=True)

        @pl.when(j == 0)
        def _():
            m_ref[...] = mj
            s_ref[...] = jnp.sum(jnp.exp(lg - mj), axis=1, keepdims=True)

        @pl.when(j > 0)
        def _():
            mold = m_ref[...]
            mnew = jnp.maximum(mold, mj)
            s_ref[...] = (s_ref[...] * jnp.exp(mold - mnew)
                          + jnp.sum(jnp.exp(lg - mnew), axis=1, keepdims=True))
            m_ref[...] = mnew

        @pl.when(j == i)
        def _():
            p_ref[...] = jnp.sum(jnp.where(is_diag, lg, 0.0), axis=1,
                                 keepdims=True)

    update(log2, m2, s2, p2)
    update(log3, m3, s3, p3)

    @pl.when((i == 0) & (j == 0))
    def _():
        acc_ref[...] = jnp.zeros((1, 1), f32)

    @pl.when(j == NB - 1)
    def _():
        part = (jnp.sum(m2[...] + jnp.log(s2[...]) - p2[...])
                + jnp.sum(m3[...] + jnp.log(s3[...]) - p3[...]))
        acc_ref[...] = acc_ref[...] + jnp.reshape(part, (1, 1))


def _run_infonce(l4, l2n, l3n):
    col = pl.BlockSpec((RB, 1), None)
    acc = pl.pallas_call(
        _infonce_kernel,
        grid=(NB, NB),
        in_specs=[
            pl.BlockSpec((RB, EMB), lambda i, j: (i, 0)),
            pl.BlockSpec((RB, EMB), lambda i, j: (j, 0)),
            pl.BlockSpec((RB, EMB), lambda i, j: (j, 0)),
        ],
        out_specs=pl.BlockSpec((1, 1), lambda i, j: (0, 0)),
        out_shape=jax.ShapeDtypeStruct((1, 1), jnp.float32),
        scratch_shapes=[pltpu.VMEM((RB, 1), jnp.float32) for _ in range(6)],
        compiler_params=pltpu.CompilerParams(
            dimension_semantics=("arbitrary", "arbitrary")),
    )(l4, l2n, l3n)
    return acc


def _bpr_kernel(g_ref, out_ref):
    u = g_ref[0]
    p = g_ref[1]
    ng = g_ref[2]
    ue = g_ref[3]
    pe = g_ref[4]
    ne = g_ref[5]
    pos = jnp.sum(u * p, axis=-1)
    neg = jnp.sum(u * ng, axis=-1)
    x = pos - neg
    logsig = jnp.minimum(x, 0.0) - jnp.log1p(jnp.exp(-jnp.abs(x)))
    cf = -jnp.sum(logsig) * (1.0 / B)
    reg = 0.5 * (jnp.sum(ue * ue) + jnp.sum(pe * pe) + jnp.sum(ne * ne)) / B
    out_ref[...] = jnp.reshape(CF_WEIGHT * cf + WEIGHT_DECAY * reg, (1, 1))


def _run_bpr(grows):
    return pl.pallas_call(
        _bpr_kernel,
        in_specs=[pl.BlockSpec((6, B, EMB), lambda: (0, 0, 0))],
        out_specs=pl.BlockSpec((1, 1), lambda: (0, 0)),
        out_shape=jax.ShapeDtypeStruct((1, 1), jnp.float32),
    )(grows)


def _timestep_embedding_const():
    ts = jnp.arange(N, dtype=jnp.float32) % STEPS
    half = D_EMB // 2
    freqs = jnp.exp(-math.log(10000.0)
                    * jnp.arange(half, dtype=jnp.float32) / half)
    a = ts[:, None] * freqs[None, :]
    return jnp.concatenate([jnp.cos(a), jnp.sin(a)], axis=-1)


def kernel(user_idx, pos_item, neg_item, edge_index, ig_edge_index,
           kg_edge_index, cf_edge_index, emb, i_Wt, i_bt, i_W1, i_b1, i_W2,
           i_b2, c_Wt, c_bt, c_W1, c_b1, c_W2, c_b2):
    i32 = jnp.int32
    f32 = jnp.float32

    eis = [edge_index, ig_edge_index, kg_edge_index, cf_edge_index]
    pad_n = EPAD - E
    pad_src = (jnp.arange(pad_n, dtype=i32) * 37) % N
    pad_dst = N + (jnp.arange(pad_n, dtype=i32) % (NP - N))
    srcs = jnp.stack([ei[0].astype(i32) for ei in eis])
    dsts = jnp.stack([ei[1].astype(i32) for ei in eis])
    srcs = jnp.concatenate(
        [srcs, jnp.broadcast_to(pad_src, (NGRAPH, pad_n))], axis=1)
    dsts = jnp.concatenate(
        [dsts, jnp.broadcast_to(pad_dst, (NGRAPH, pad_n))], axis=1)
    goff = (jnp.arange(NGRAPH, dtype=i32) * NP)[:, None]
    dsts_flat = dsts.reshape(-1)
    arr_s = (srcs + goff).reshape(NGRAPH, NS, CHUNKS, CH)
    arr_d = dsts.reshape(NGRAPH, NS, CHUNKS, CH)
    idxcat = jnp.stack([arr_s, arr_d], axis=3).reshape(-1, CH)

    zeros1 = jnp.zeros((NP,), f32)
    zeros2 = jnp.zeros((NP, EMB), f32)

    deg_flat = _sc_degree(dsts_flat, zeros1)
    degp = deg_flat.reshape(NGRAPH, NP).T

    embp = jnp.concatenate([emb.astype(f32),
                            jnp.zeros((NP - N, EMB), f32)], axis=0)
    y1 = _scale_emb(degp, embp)
    s1 = _sc_propagate(idxcat, y1, zeros2)
    y2 = _scale_s1(degp, s1)
    s2 = _sc_propagate(idxcat, y2, zeros2)

    s1r = s1.reshape(NGRAPH, NP, EMB)
    s2r = s2.reshape(NGRAPH, NP, EMB)
    s1g = [s1r[g] for g in range(NGRAPH)]
    s2g = [s2r[g] for g in range(NGRAPH)]

    betas = NOISE_SCALE * jnp.linspace(NOISE_MIN, NOISE_MAX, STEPS)
    ab = jnp.cumprod(1.0 - betas)
    ts = jnp.arange(N) % STEPS
    abt = ab[ts][:, None].astype(f32)
    sab = jnp.sqrt(abt)
    sbb = jnp.sqrt(1.0 - abt)
    n1 = jax.random.normal(jax.random.key(1), (N, EMB), dtype=f32)
    n2 = jax.random.normal(jax.random.key(2), (N, EMB), dtype=f32)
    te = _timestep_embedding_const()

    wts = [i_Wt, i_bt.reshape(1, D_EMB), i_W1[:EMB], i_W1[EMB:],
           i_b1.reshape(1, EMB), i_W2, i_b2.reshape(1, EMB),
           c_Wt, c_bt.reshape(1, D_EMB), c_W1[:EMB], c_W1[EMB:],
           c_b1.reshape(1, EMB), c_W2, c_b2.reshape(1, EMB)]

    all_layer, l4, l2n, l3n, diff_acc = _run_main(
        emb.astype(f32), s1g, s2g, degp[:N], n1, n2, sab, sbb, te, wts)

    nce_acc = _run_infonce(l4, l2n, l3n)

    tables = jnp.concatenate([all_layer, emb.astype(f32)], axis=0)
    idx_all = jnp.concatenate([
        user_idx.astype(i32), pos_item.astype(i32), neg_item.astype(i32),
        user_idx.astype(i32) + N, pos_item.astype(i32) + N,
        neg_item.astype(i32) + N])
    grows = _sc_gather_rows(idx_all, tables).reshape(6, B, EMB)
    bpr = _run_bpr(grows)

    diff_loss = diff_acc[0, 0] * (1.0 / N)
    gcl = nce_acc[0, 0] * (1.0 / N)
    return bpr[0, 0] + diff_loss + GCL_WEIGHT * gcl


# speedup vs baseline: 15.0674x; 1.0002x over previous
import functools
import math

import jax
import jax.numpy as jnp
from jax import lax
from jax.experimental import pallas as pl
from jax.experimental.pallas import tpu as pltpu
from jax.experimental.pallas import tpu_sc as plsc

N = 10000
EMB = 128
E = 320000
NGRAPH = 4
LAYERS = 2
STEPS = 5
NOISE_SCALE = 0.1
NOISE_MIN = 0.0001
NOISE_MAX = 0.02
CF_WEIGHT = 1.0
WEIGHT_DECAY = 0.0001
GCL_WEIGHT = 0.1
GCL_TEMP = 0.2
D_EMB = 10
B = 4096

NC = 2
NS = 16
CH = 128
NBUF = 4
NBUF_P = 2
CHUNKS = 160
EPT = CHUNKS * CH
EPAD = NS * EPT
NP = 10240
RPT = NP // NS

RB = 1000
NB = N // RB


_sc_mesh = plsc.VectorSubcoreMesh(core_axis_name="c", subcore_axis_name="s")


@functools.partial(
    pl.kernel,
    out_type=jax.ShapeDtypeStruct((NGRAPH * NP,), jnp.float32),
    mesh=_sc_mesh,
    scratch_types=[
        pltpu.VMEM_SHARED((NP,), jnp.float32),
        pltpu.VMEM_SHARED((NP,), jnp.float32),
        pltpu.VMEM((CH,), jnp.float32),
        *[pltpu.VMEM((CH,), jnp.int32) for _ in range(NBUF)],
        *[pltpu.SemaphoreType.DMA for _ in range(NBUF)],
    ],
)
def _sc_degree(dsts, zeros1, deg_out, acc0, acc1, ones_v, *rest):
    idx = rest[:NBUF]
    sem = rest[NBUF:]
    cid = lax.axis_index("c")
    sid = lax.axis_index("s")
    accs = [acc0, acc1]
    for j in range(CH // 16):
        ones_v[pl.ds(j * 16, 16)] = jnp.ones((16,), jnp.float32)
    for gl in range(2):
        pltpu.sync_copy(zeros1.at[pl.ds(sid * RPT, RPT)],
                        accs[gl].at[pl.ds(sid * RPT, RPT)])
    plsc.subcore_barrier()
    for gl in range(2):
        g = cid * 2 + gl
        ebase = g * EPAD + sid * EPT

        def body(it, _, gl=gl, ebase=ebase):
            cps = []
            for b in range(NBUF):
                off = pl.multiple_of(ebase + (it * NBUF + b) * CH, CH)
                cps.append(pltpu.async_copy(
                    dsts.at[pl.ds(off, CH)], idx[b], sem[b]))
            for b in range(NBUF):
                cps[b].wait()
                pltpu.sync_copy(ones_v, accs[gl].at[idx[b]], add=True)
            return 0

        lax.fori_loop(0, CHUNKS // NBUF, body, 0)
    plsc.subcore_barrier()
    for gl in range(2):
        g = cid * 2 + gl
        off = pl.multiple_of(g * NP + sid * RPT, 8)
        pltpu.sync_copy(accs[gl].at[pl.ds(sid * RPT, RPT)],
                        deg_out.at[pl.ds(off, RPT)])


@functools.partial(
    pl.kernel,
    out_type=jax.ShapeDtypeStruct((NGRAPH * NP, EMB), jnp.float32),
    mesh=_sc_mesh,
    scratch_types=[
        pltpu.VMEM_SHARED((NP, EMB), jnp.float32),
        pltpu.VMEM((8, CH), jnp.int32),
        pltpu.VMEM((8, CH), jnp.int32),
        *[pltpu.VMEM((CH, EMB), jnp.float32) for _ in range(2)],
        *[pltpu.SemaphoreType.DMA for _ in range(6)],
    ],
)
def _sc_propagate(idxcat, table, zeros2, out, acc, bx, by, r0, r1,
                  smx, smy, sg0, sg1, ss0, ss1):
    cid = lax.axis_index("c")
    sid = lax.axis_index("s")
    rows = [r0, r1]
    sem_g = [sg0, sg1]
    sem_s = [ss0, ss1]
    nbody = CHUNKS // 8

    def fire_idx(buf, sem, g, body_ix, half):
        base = (g * NS + sid) * (2 * CHUNKS)
        off = base + body_ix * 16 + half * 8
        return pltpu.async_copy(idxcat.at[pl.ds(off, 8)], buf, sem)

    def fire_g(ib, j, b):
        return pltpu.async_copy(table.at[ib.at[2 * j]], rows[b], sem_g[b])

    def fire_s(ib, j, b):
        return pltpu.async_copy(rows[b], acc.at[ib.at[2 * j + 1]],
                                sem_s[b], add=True)

    def wait(sem, ref):
        if ref is bx or ref is by:
            dummy = idxcat.at[pl.ds(0, 8)]
        else:
            dummy = table.at[pl.ds(0, CH)]
        pltpu.make_async_copy(dummy, ref, sem).wait()

    for gl in range(2):
        g = cid * 2 + gl
        pltpu.sync_copy(zeros2.at[pl.ds(sid * RPT, RPT)],
                        acc.at[pl.ds(sid * RPT, RPT)])
        plsc.subcore_barrier()

        fire_idx(bx, smx, g, 0, 0).wait()
        fire_idx(by, smy, g, 0, 1)
        fire_g(bx, 0, 0)
        fire_g(bx, 1, 1)

        def body(k, _, g=g):
            last = k == nbody - 1
            wait(sem_g[0], rows[0]); fire_s(bx, 0, 0)
            wait(sem_g[1], rows[1]); fire_s(bx, 1, 1)
            wait(sem_s[0], rows[0]); fire_g(bx, 2, 0)
            wait(sem_s[1], rows[1]); fire_g(bx, 3, 1)
            wait(smy, by)
            wait(sem_g[0], rows[0]); fire_s(bx, 2, 0)
            wait(sem_g[1], rows[1]); fire_s(bx, 3, 1)
            wait(sem_s[0], rows[0]); fire_g(by, 0, 0)
            wait(sem_s[1], rows[1]); fire_g(by, 1, 1)

            @pl.when(jnp.logical_not(last))
            def _():
                fire_idx(bx, smx, g, k + 1, 0)

            wait(sem_g[0], rows[0]); fire_s(by, 0, 0)
            wait(sem_g[1], rows[1]); fire_s(by, 1, 1)
            wait(sem_s[0], rows[0]); fire_g(by, 2, 0)
            wait(sem_s[1], rows[1]); fire_g(by, 3, 1)
            wait(sem_g[0], rows[0]); fire_s(by, 2, 0)
            wait(sem_g[1], rows[1]); fire_s(by, 3, 1)
            wait(sem_s[0], rows[0])

            @pl.when(jnp.logical_not(last))
            def _():
                wait(smx, bx)
                fire_g(bx, 0, 0)

            wait(sem_s[1], rows[1])

            @pl.when(jnp.logical_not(last))
            def _():
                fire_g(bx, 1, 1)
                fire_idx(by, smy, g, k + 1, 1)

            return 0

        lax.fori_loop(0, nbody, body, 0)
        plsc.subcore_barrier()
        off = g * NP + sid * RPT
        pltpu.sync_copy(acc.at[pl.ds(sid * RPT, RPT)],
                        out.at[pl.ds(off, RPT)])
        plsc.subcore_barrier()


NIDX = 6 * B
GPW = NIDX // (NC * NS)
GCH = GPW // CH


@functools.partial(
    pl.kernel,
    out_type=jax.ShapeDtypeStruct((NIDX, EMB), jnp.float32),
    mesh=_sc_mesh,
    scratch_types=[
        *[pltpu.VMEM((CH,), jnp.int32) for _ in range(2)],
        *[pltpu.VMEM((CH, EMB), jnp.float32) for _ in range(2)],
        *[pltpu.SemaphoreType.DMA for _ in range(4)],
    ],
)
def _sc_gather_rows(idx_all, tables, out, i0, i1, r0, r1, si0, si1, sg0, sg1):
    cid = lax.axis_index("c")
    sid = lax.axis_index("s")
    wid = sid * NC + cid
    base = wid * GPW
    idx = [i0, i1]
    rows = [r0, r1]
    sem_i = [si0, si1]
    sem_g = [sg0, sg1]
    for k in range(GCH):
        b = k % 2
        off = pl.multiple_of(base + k * CH, CH)
        pltpu.async_copy(idx_all.at[pl.ds(off, CH)], idx[b], sem_i[b]).wait()
        pltpu.async_copy(tables.at[idx[b]], rows[b], sem_g[b]).wait()
        pltpu.sync_copy(rows[b], out.at[pl.ds(off, CH)])


def _dinv_of(degcol):
    return jnp.where(degcol > 0.0, lax.rsqrt(jnp.maximum(degcol, 1e-30)), 0.0)


def _scale_table_kernel(degp_ref, x_ref, out_ref, *, power):
    g = pl.program_id(0)
    degs = degp_ref[...]
    col = jnp.zeros_like(degs[:, 0:1])
    for k in range(NGRAPH):
        col = col + jnp.where(g == k, degs[:, k:k + 1], 0.0)
    d = _dinv_of(col)
    w = d * d if power == 2 else d
    out_ref[...] = w * x_ref[...]


def _scale_emb(degp, embp):
    out = pl.pallas_call(
        functools.partial(_scale_table_kernel, power=1),
        grid=(NGRAPH, NP // 1024),
        in_specs=[
            pl.BlockSpec((1024, NGRAPH), lambda g, i: (i, 0)),
            pl.BlockSpec((1024, EMB), lambda g, i: (i, 0)),
        ],
        out_specs=pl.BlockSpec((1024, EMB), lambda g, i: (g * (NP // 1024) + i, 0)),
        out_shape=jax.ShapeDtypeStruct((NGRAPH * NP, EMB), jnp.float32),
    )(degp, embp)
    return out


def _scale_s1(degp, s1):
    out = pl.pallas_call(
        functools.partial(_scale_table_kernel, power=2),
        grid=(NGRAPH, NP // 1024),
        in_specs=[
            pl.BlockSpec((1024, NGRAPH), lambda g, i: (i, 0)),
            pl.BlockSpec((1024, EMB), lambda g, i: (g * (NP // 1024) + i, 0)),
        ],
        out_specs=pl.BlockSpec((1024, EMB), lambda g, i: (g * (NP // 1024) + i, 0)),
        out_shape=jax.ShapeDtypeStruct((NGRAPH * NP, EMB), jnp.float32),
    )(degp, s1)
    return out


def _rownorm(x):
    n = jnp.sqrt(jnp.sum(x * x, axis=-1, keepdims=True))
    return x / jnp.maximum(n, 1e-12)


def _main_kernel(emb_ref, s1_0, s1_1, s1_2, s1_3, s2_0, s2_1, s2_2, s2_3,
                 degp_ref, n1_ref, n2_ref, sab_ref, sbb_ref, te_ref,
                 iWt, ibt, iW1a, iW1b, ib1, iW2, ib2,
                 cWt, cbt, cW1a, cW1b, cb1, cW2, cb2,
                 all_ref, l4_ref, l2n_ref, l3n_ref, acc_ref):
    i = pl.program_id(0)
    deg = degp_ref[...]
    emb = emb_ref[...]

    def allg(g, s1b, s2b):
        d = _dinv_of(deg[:, g:g + 1])
        return (emb + d * (s1b[...] + s2b[...])) * (1.0 / 3.0)

    all_layer = allg(0, s1_0, s2_0)
    l2 = allg(1, s1_1, s2_1)
    l3 = allg(2, s1_2, s2_2)
    l1n = _rownorm(allg(3, s1_3, s2_3))

    f32 = jnp.float32

    def mlp(x_t, Wt, bt, W1a, W1b, b1, W2, b2):
        temb = jnp.dot(te_ref[...], Wt[...], preferred_element_type=f32) + bt[...]
        h = jnp.tanh(jnp.dot(x_t, W1a[...], preferred_element_type=f32)
                     + jnp.dot(temb, W1b[...], preferred_element_type=f32)
                     + b1[...])
        return jnp.dot(h, W2[...], preferred_element_type=f32) + b2[...]

    sab = sab_ref[...]
    sbb = sbb_ref[...]
    x_t2 = sab * l2 + sbb * n1_ref[...]
    x_t3 = sab * l3 + sbb * n2_ref[...]
    d2 = mlp(x_t2, iWt, ibt, iW1a, iW1b, ib1, iW2, ib2)
    d3 = mlp(x_t3, cWt, cbt, cW1a, cW1b, cb1, cW2, cb2)

    part = (jnp.sum((l1n - d2) ** 2) + jnp.sum((l1n - d3) ** 2)) * (1.0 / EMB)

    all_ref[...] = all_layer
    l4_ref[...] = _rownorm(all_layer)
    l2n_ref[...] = _rownorm(l2 + d2)
    l3n_ref[...] = _rownorm(l3 + d3)

    @pl.when(i == 0)
    def _():
        acc_ref[...] = jnp.zeros((1, 1), f32)

    acc_ref[...] = acc_ref[...] + jnp.reshape(part, (1, 1))


def _run_main(embk, s1r, s2r, degp, n1, n2, sab, sbb, te, wts):
    rowspec = pl.BlockSpec((RB, EMB), lambda i: (i, 0))
    wspecs = []
    for w in wts:
        wspecs.append(pl.BlockSpec(w.shape, lambda i: (0,) * w.ndim))
    outs = pl.pallas_call(
        _main_kernel,
        grid=(NB,),
        in_specs=[
            rowspec,
            *[rowspec] * 8,
            pl.BlockSpec((RB, NGRAPH), lambda i: (i, 0)),
            rowspec, rowspec,
            pl.BlockSpec((RB, 1), lambda i: (i, 0)),
            pl.BlockSpec((RB, 1), lambda i: (i, 0)),
            pl.BlockSpec((RB, D_EMB), lambda i: (i, 0)),
            *wspecs,
        ],
        out_specs=[
            rowspec, rowspec, rowspec, rowspec,
            pl.BlockSpec((1, 1), lambda i: (0, 0)),
        ],
        out_shape=[
            jax.ShapeDtypeStruct((N, EMB), jnp.float32),
            jax.ShapeDtypeStruct((N, EMB), jnp.float32),
            jax.ShapeDtypeStruct((N, EMB), jnp.float32),
            jax.ShapeDtypeStruct((N, EMB), jnp.float32),
            jax.ShapeDtypeStruct((1, 1), jnp.float32),
        ],
        compiler_params=pltpu.CompilerParams(
            dimension_semantics=("arbitrary",)),
    )(embk, s1r[0], s1r[1], s1r[2], s1r[3], s2r[0], s2r[1], s2r[2], s2r[3],
      degp, n1, n2, sab, sbb, te, *wts)
    return outs


def _infonce_kernel(l4_ref, b2_ref, b3_ref, acc_ref,
                    m2, s2, p2, m3, s3, p3):
    i = pl.program_id(0)
    j = pl.program_id(1)
    a = l4_ref[...] * (1.0 / GCL_TEMP)
    f32 = jnp.float32
    ab = a.astype(jnp.bfloat16)
    dn = (((1,), (1,)), ((), ()))
    log2 = lax.dot_general(ab, b2_ref[...].astype(jnp.bfloat16), dn,
                           preferred_element_type=f32)
    log3 = lax.dot_general(ab, b3_ref[...].astype(jnp.bfloat16), dn,
                           preferred_element_type=f32)

    def update(lg, bref, m_ref, s_ref, p_ref):
        mj = jnp.max(lg, axis=1, keepdims=True)

        @pl.when(j == 0)
        def _():
            m_ref[...] = mj
            s_ref[...] = jnp.sum(jnp.exp(lg - mj), axis=1, keepdims=True)

        @pl.when(j > 0)
        def _():
            mold = m_ref[...]
            mnew = jnp.maximum(mold, mj)
            s_ref[...] = (s_ref[...] * jnp.exp(mold - mnew)
                          + jnp.sum(jnp.exp(lg - mnew), axis=1, keepdims=True))
            m_ref[...] = mnew

        @pl.when(j == i)
        def _():
            p_ref[...] = jnp.sum(a * bref[...], axis=1, keepdims=True)

    update(log2, b2_ref, m2, s2, p2)
    update(log3, b3_ref, m3, s3, p3)

    @pl.when((i == 0) & (j == 0))
    def _():
        acc_ref[...] = jnp.zeros((1, 1), f32)

    @pl.when(j == NB - 1)
    def _():
        part = (jnp.sum(m2[...] + jnp.log(s2[...]) - p2[...])
                + jnp.sum(m3[...] + jnp.log(s3[...]) - p3[...]))
        acc_ref[...] = acc_ref[...] + jnp.reshape(part, (1, 1))


def _run_infonce(l4, l2n, l3n):
    col = pl.BlockSpec((RB, 1), None)
    acc = pl.pallas_call(
        _infonce_kernel,
        grid=(NB, NB),
        in_specs=[
            pl.BlockSpec((RB, EMB), lambda i, j: (i, 0)),
            pl.BlockSpec((RB, EMB), lambda i, j: (j, 0)),
            pl.BlockSpec((RB, EMB), lambda i, j: (j, 0)),
        ],
        out_specs=pl.BlockSpec((1, 1), lambda i, j: (0, 0)),
        out_shape=jax.ShapeDtypeStruct((1, 1), jnp.float32),
        scratch_shapes=[pltpu.VMEM((RB, 1), jnp.float32) for _ in range(6)],
        compiler_params=pltpu.CompilerParams(
            dimension_semantics=("arbitrary", "arbitrary")),
    )(l4, l2n, l3n)
    return acc


def _bpr_kernel(g_ref, out_ref):
    u = g_ref[0]
    p = g_ref[1]
    ng = g_ref[2]
    ue = g_ref[3]
    pe = g_ref[4]
    ne = g_ref[5]
    pos = jnp.sum(u * p, axis=-1)
    neg = jnp.sum(u * ng, axis=-1)
    x = pos - neg
    logsig = jnp.minimum(x, 0.0) - jnp.log1p(jnp.exp(-jnp.abs(x)))
    cf = -jnp.sum(logsig) * (1.0 / B)
    reg = 0.5 * (jnp.sum(ue * ue) + jnp.sum(pe * pe) + jnp.sum(ne * ne)) / B
    out_ref[...] = jnp.reshape(CF_WEIGHT * cf + WEIGHT_DECAY * reg, (1, 1))


def _run_bpr(grows):
    return pl.pallas_call(
        _bpr_kernel,
        in_specs=[pl.BlockSpec((6, B, EMB), lambda: (0, 0, 0))],
        out_specs=pl.BlockSpec((1, 1), lambda: (0, 0)),
        out_shape=jax.ShapeDtypeStruct((1, 1), jnp.float32),
    )(grows)


def _timestep_embedding_const():
    ts = jnp.arange(N, dtype=jnp.float32) % STEPS
    half = D_EMB // 2
    freqs = jnp.exp(-math.log(10000.0)
                    * jnp.arange(half, dtype=jnp.float32) / half)
    a = ts[:, None] * freqs[None, :]
    return jnp.concatenate([jnp.cos(a), jnp.sin(a)], axis=-1)


def kernel(user_idx, pos_item, neg_item, edge_index, ig_edge_index,
           kg_edge_index, cf_edge_index, emb, i_Wt, i_bt, i_W1, i_b1, i_W2,
           i_b2, c_Wt, c_bt, c_W1, c_b1, c_W2, c_b2):
    i32 = jnp.int32
    f32 = jnp.float32

    eis = [edge_index, ig_edge_index, kg_edge_index, cf_edge_index]
    pad_n = EPAD - E
    pad_src = (jnp.arange(pad_n, dtype=i32) * 37) % N
    pad_dst = N + (jnp.arange(pad_n, dtype=i32) % (NP - N))
    srcs = jnp.stack([ei[0].astype(i32) for ei in eis])
    dsts = jnp.stack([ei[1].astype(i32) for ei in eis])
    srcs = jnp.concatenate(
        [srcs, jnp.broadcast_to(pad_src, (NGRAPH, pad_n))], axis=1)
    dsts = jnp.concatenate(
        [dsts, jnp.broadcast_to(pad_dst, (NGRAPH, pad_n))], axis=1)
    goff = (jnp.arange(NGRAPH, dtype=i32) * NP)[:, None]
    dsts_flat = dsts.reshape(-1)
    arr_s = (srcs + goff).reshape(NGRAPH, NS, CHUNKS, CH)
    arr_d = dsts.reshape(NGRAPH, NS, CHUNKS, CH)
    idxcat = jnp.stack([arr_s, arr_d], axis=3).reshape(-1, CH)

    zeros1 = jnp.zeros((NP,), f32)
    zeros2 = jnp.zeros((NP, EMB), f32)

    deg_flat = _sc_degree(dsts_flat, zeros1)
    degp = deg_flat.reshape(NGRAPH, NP).T

    embp = jnp.concatenate([emb.astype(f32),
                            jnp.zeros((NP - N, EMB), f32)], axis=0)
    y1 = _scale_emb(degp, embp)
    s1 = _sc_propagate(idxcat, y1, zeros2)
    y2 = _scale_s1(degp, s1)
    s2 = _sc_propagate(idxcat, y2, zeros2)

    s1r = s1.reshape(NGRAPH, NP, EMB)
    s2r = s2.reshape(NGRAPH, NP, EMB)
    s1g = [s1r[g] for g in range(NGRAPH)]
    s2g = [s2r[g] for g in range(NGRAPH)]

    betas = NOISE_SCALE * jnp.linspace(NOISE_MIN, NOISE_MAX, STEPS)
    ab = jnp.cumprod(1.0 - betas)
    ts = jnp.arange(N) % STEPS
    abt = ab[ts][:, None].astype(f32)
    sab = jnp.sqrt(abt)
    sbb = jnp.sqrt(1.0 - abt)
    n1 = jax.random.normal(jax.random.key(1), (N, EMB), dtype=f32)
    n2 = jax.random.normal(jax.random.key(2), (N, EMB), dtype=f32)
    te = _timestep_embedding_const()

    wts = [i_Wt, i_bt.reshape(1, D_EMB), i_W1[:EMB], i_W1[EMB:],
           i_b1.reshape(1, EMB), i_W2, i_b2.reshape(1, EMB),
           c_Wt, c_bt.reshape(1, D_EMB), c_W1[:EMB], c_W1[EMB:],
           c_b1.reshape(1, EMB), c_W2, c_b2.reshape(1, EMB)]

    all_layer, l4, l2n, l3n, diff_acc = _run_main(
        emb.astype(f32), s1g, s2g, degp[:N], n1, n2, sab, sbb, te, wts)

    nce_acc = _run_infonce(l4, l2n, l3n)

    tables = jnp.concatenate([all_layer, emb.astype(f32)], axis=0)
    idx_all = jnp.concatenate([
        user_idx.astype(i32), pos_item.astype(i32), neg_item.astype(i32),
        user_idx.astype(i32) + N, pos_item.astype(i32) + N,
        neg_item.astype(i32) + N])
    grows = _sc_gather_rows(idx_all, tables).reshape(6, B, EMB)
    bpr = _run_bpr(grows)

    diff_loss = diff_acc[0, 0] * (1.0 / N)
    gcl = nce_acc[0, 0] * (1.0 / N)
    return bpr[0, 0] + diff_loss + GCL_WEIGHT * gcl


# speedup vs baseline: 15.9653x; 1.0596x over previous
import functools
import math

import jax
import jax.numpy as jnp
from jax import lax
from jax.experimental import pallas as pl
from jax.experimental.pallas import tpu as pltpu
from jax.experimental.pallas import tpu_sc as plsc

N = 10000
EMB = 128
E = 320000
NGRAPH = 4
LAYERS = 2
STEPS = 5
NOISE_SCALE = 0.1
NOISE_MIN = 0.0001
NOISE_MAX = 0.02
CF_WEIGHT = 1.0
WEIGHT_DECAY = 0.0001
GCL_WEIGHT = 0.1
GCL_TEMP = 0.2
D_EMB = 10
B = 4096

NC = 2
NS = 16
CH = 128
NBUF = 4
NBUF_P = 2
CHUNKS = 160
EPT = CHUNKS * CH
EPAD = NS * EPT
NP = 10240
RPT = NP // NS

RB = 1000
NB = N // RB


_sc_mesh = plsc.VectorSubcoreMesh(core_axis_name="c", subcore_axis_name="s")


@functools.partial(
    pl.kernel,
    out_type=jax.ShapeDtypeStruct((NGRAPH * NP,), jnp.float32),
    mesh=_sc_mesh,
    scratch_types=[
        pltpu.VMEM_SHARED((NP,), jnp.float32),
        pltpu.VMEM_SHARED((NP,), jnp.float32),
        pltpu.VMEM((CH,), jnp.float32),
        *[pltpu.VMEM((CH,), jnp.int32) for _ in range(NBUF)],
        *[pltpu.SemaphoreType.DMA for _ in range(NBUF)],
    ],
)
def _sc_degree(dsts, zeros1, deg_out, acc0, acc1, ones_v, *rest):
    idx = rest[:NBUF]
    sem = rest[NBUF:]
    cid = lax.axis_index("c")
    sid = lax.axis_index("s")
    accs = [acc0, acc1]
    for j in range(CH // 16):
        ones_v[pl.ds(j * 16, 16)] = jnp.ones((16,), jnp.float32)
    for gl in range(2):
        pltpu.sync_copy(zeros1.at[pl.ds(sid * RPT, RPT)],
                        accs[gl].at[pl.ds(sid * RPT, RPT)])
    plsc.subcore_barrier()
    for gl in range(2):
        g = cid * 2 + gl
        ebase = g * EPAD + sid * EPT

        def body(it, _, gl=gl, ebase=ebase):
            cps = []
            for b in range(NBUF):
                off = pl.multiple_of(ebase + (it * NBUF + b) * CH, CH)
                cps.append(pltpu.async_copy(
                    dsts.at[pl.ds(off, CH)], idx[b], sem[b]))
            for b in range(NBUF):
                cps[b].wait()
                pltpu.sync_copy(ones_v, accs[gl].at[idx[b]], add=True)
            return 0

        lax.fori_loop(0, CHUNKS // NBUF, body, 0)
    plsc.subcore_barrier()
    for gl in range(2):
        g = cid * 2 + gl
        off = pl.multiple_of(g * NP + sid * RPT, 8)
        pltpu.sync_copy(accs[gl].at[pl.ds(sid * RPT, RPT)],
                        deg_out.at[pl.ds(off, RPT)])


@functools.partial(
    pl.kernel,
    out_type=jax.ShapeDtypeStruct((NGRAPH * NP, EMB), jnp.float32),
    mesh=_sc_mesh,
    scratch_types=[
        pltpu.VMEM_SHARED((NP, EMB), jnp.float32),
        pltpu.VMEM((8, CH), jnp.int32),
        pltpu.VMEM((8, CH), jnp.int32),
        *[pltpu.VMEM((CH, EMB), jnp.float32) for _ in range(2)],
        *[pltpu.SemaphoreType.DMA for _ in range(6)],
    ],
)
def _sc_propagate(idxcat, table, zeros2, out, acc, bx, by, r0, r1,
                  smx, smy, sg0, sg1, ss0, ss1):
    cid = lax.axis_index("c")
    sid = lax.axis_index("s")
    rows = [r0, r1]
    sem_g = [sg0, sg1]
    sem_s = [ss0, ss1]
    nbody = CHUNKS // 8

    def fire_idx(buf, sem, g, body_ix, half):
        base = (g * NS + sid) * (2 * CHUNKS)
        off = base + body_ix * 16 + half * 8
        return pltpu.async_copy(idxcat.at[pl.ds(off, 8)], buf, sem)

    def fire_g(ib, j, b):
        return pltpu.async_copy(table.at[ib.at[2 * j]], rows[b], sem_g[b])

    def fire_s(ib, j, b):
        return pltpu.async_copy(rows[b], acc.at[ib.at[2 * j + 1]],
                                sem_s[b], add=True)

    def wait(sem, ref):
        if ref is bx or ref is by:
            dummy = idxcat.at[pl.ds(0, 8)]
        else:
            dummy = table.at[pl.ds(0, CH)]
        pltpu.make_async_copy(dummy, ref, sem).wait()

    for gl in range(2):
        g = cid * 2 + gl
        pltpu.sync_copy(zeros2.at[pl.ds(sid * RPT, RPT)],
                        acc.at[pl.ds(sid * RPT, RPT)])
        plsc.subcore_barrier()

        fire_idx(bx, smx, g, 0, 0).wait()
        fire_idx(by, smy, g, 0, 1)
        fire_g(bx, 0, 0)
        fire_g(bx, 1, 1)

        def body(k, _, g=g):
            last = k == nbody - 1
            wait(sem_g[0], rows[0]); fire_s(bx, 0, 0)
            wait(sem_g[1], rows[1]); fire_s(bx, 1, 1)
            wait(sem_s[0], rows[0]); fire_g(bx, 2, 0)
            wait(sem_s[1], rows[1]); fire_g(bx, 3, 1)
            wait(smy, by)
            wait(sem_g[0], rows[0]); fire_s(bx, 2, 0)
            wait(sem_g[1], rows[1]); fire_s(bx, 3, 1)
            wait(sem_s[0], rows[0]); fire_g(by, 0, 0)
            wait(sem_s[1], rows[1]); fire_g(by, 1, 1)

            @pl.when(jnp.logical_not(last))
            def _():
                fire_idx(bx, smx, g, k + 1, 0)

            wait(sem_g[0], rows[0]); fire_s(by, 0, 0)
            wait(sem_g[1], rows[1]); fire_s(by, 1, 1)
            wait(sem_s[0], rows[0]); fire_g(by, 2, 0)
            wait(sem_s[1], rows[1]); fire_g(by, 3, 1)
            wait(sem_g[0], rows[0]); fire_s(by, 2, 0)
            wait(sem_g[1], rows[1]); fire_s(by, 3, 1)
            wait(sem_s[0], rows[0])

            @pl.when(jnp.logical_not(last))
            def _():
                wait(smx, bx)
                fire_g(bx, 0, 0)

            wait(sem_s[1], rows[1])

            @pl.when(jnp.logical_not(last))
            def _():
                fire_g(bx, 1, 1)
                fire_idx(by, smy, g, k + 1, 1)

            return 0

        lax.fori_loop(0, nbody, body, 0)
        plsc.subcore_barrier()
        off = g * NP + sid * RPT
        pltpu.sync_copy(acc.at[pl.ds(sid * RPT, RPT)],
                        out.at[pl.ds(off, RPT)])
        plsc.subcore_barrier()


NIDX = 6 * B
GPW = NIDX // (NC * NS)
GCH = GPW // CH


@functools.partial(
    pl.kernel,
    out_type=jax.ShapeDtypeStruct((NIDX, EMB), jnp.float32),
    mesh=_sc_mesh,
    scratch_types=[
        *[pltpu.VMEM((CH,), jnp.int32) for _ in range(2)],
        *[pltpu.VMEM((CH, EMB), jnp.float32) for _ in range(2)],
        *[pltpu.SemaphoreType.DMA for _ in range(4)],
    ],
)
def _sc_gather_rows(idx_all, tables, out, i0, i1, r0, r1, si0, si1, sg0, sg1):
    cid = lax.axis_index("c")
    sid = lax.axis_index("s")
    wid = sid * NC + cid
    base = wid * GPW
    idx = [i0, i1]
    rows = [r0, r1]
    sem_i = [si0, si1]
    sem_g = [sg0, sg1]
    for k in range(GCH):
        b = k % 2
        off = pl.multiple_of(base + k * CH, CH)
        pltpu.async_copy(idx_all.at[pl.ds(off, CH)], idx[b], sem_i[b]).wait()
        pltpu.async_copy(tables.at[idx[b]], rows[b], sem_g[b]).wait()
        pltpu.sync_copy(rows[b], out.at[pl.ds(off, CH)])


def _dinv_of(degcol):
    return jnp.where(degcol > 0.0, lax.rsqrt(jnp.maximum(degcol, 1e-30)), 0.0)


def _scale_table_kernel(degp_ref, x_ref, out_ref, *, power):
    g = pl.program_id(0)
    degs = degp_ref[...]
    col = jnp.zeros_like(degs[:, 0:1])
    for k in range(NGRAPH):
        col = col + jnp.where(g == k, degs[:, k:k + 1], 0.0)
    d = _dinv_of(col)
    w = d * d if power == 2 else d
    out_ref[...] = w * x_ref[...]


def _scale_emb(degp, embp):
    out = pl.pallas_call(
        functools.partial(_scale_table_kernel, power=1),
        grid=(NGRAPH, NP // 1024),
        in_specs=[
            pl.BlockSpec((1024, NGRAPH), lambda g, i: (i, 0)),
            pl.BlockSpec((1024, EMB), lambda g, i: (i, 0)),
        ],
        out_specs=pl.BlockSpec((1024, EMB), lambda g, i: (g * (NP // 1024) + i, 0)),
        out_shape=jax.ShapeDtypeStruct((NGRAPH * NP, EMB), jnp.float32),
    )(degp, embp)
    return out


def _scale_s1(degp, s1):
    out = pl.pallas_call(
        functools.partial(_scale_table_kernel, power=2),
        grid=(NGRAPH, NP // 1024),
        in_specs=[
            pl.BlockSpec((1024, NGRAPH), lambda g, i: (i, 0)),
            pl.BlockSpec((1024, EMB), lambda g, i: (g * (NP // 1024) + i, 0)),
        ],
        out_specs=pl.BlockSpec((1024, EMB), lambda g, i: (g * (NP // 1024) + i, 0)),
        out_shape=jax.ShapeDtypeStruct((NGRAPH * NP, EMB), jnp.float32),
    )(degp, s1)
    return out


def _rownorm(x):
    n = jnp.sqrt(jnp.sum(x * x, axis=-1, keepdims=True))
    return x / jnp.maximum(n, 1e-12)


def _main_kernel(emb_ref, s1_0, s1_1, s1_2, s1_3, s2_0, s2_1, s2_2, s2_3,
                 degp_ref, n1_ref, n2_ref, sab_ref, sbb_ref, te_ref,
                 iWt, ibt, iW1a, iW1b, ib1, iW2, ib2,
                 cWt, cbt, cW1a, cW1b, cb1, cW2, cb2,
                 all_ref, l4_ref, l2n_ref, l3n_ref, acc_ref):
    i = pl.program_id(0)
    deg = degp_ref[...]
    emb = emb_ref[...]

    def allg(g, s1b, s2b):
        d = _dinv_of(deg[:, g:g + 1])
        return (emb + d * (s1b[...] + s2b[...])) * (1.0 / 3.0)

    all_layer = allg(0, s1_0, s2_0)
    l2 = allg(1, s1_1, s2_1)
    l3 = allg(2, s1_2, s2_2)
    l1n = _rownorm(allg(3, s1_3, s2_3))

    f32 = jnp.float32

    def mlp(x_t, Wt, bt, W1a, W1b, b1, W2, b2):
        temb = jnp.dot(te_ref[...], Wt[...], preferred_element_type=f32) + bt[...]
        h = jnp.tanh(jnp.dot(x_t, W1a[...], preferred_element_type=f32)
                     + jnp.dot(temb, W1b[...], preferred_element_type=f32)
                     + b1[...])
        return jnp.dot(h, W2[...], preferred_element_type=f32) + b2[...]

    sab = sab_ref[...]
    sbb = sbb_ref[...]
    x_t2 = sab * l2 + sbb * n1_ref[...]
    x_t3 = sab * l3 + sbb * n2_ref[...]
    d2 = mlp(x_t2, iWt, ibt, iW1a, iW1b, ib1, iW2, ib2)
    d3 = mlp(x_t3, cWt, cbt, cW1a, cW1b, cb1, cW2, cb2)

    part = (jnp.sum((l1n - d2) ** 2) + jnp.sum((l1n - d3) ** 2)) * (1.0 / EMB)

    all_ref[...] = all_layer
    l4_ref[...] = _rownorm(all_layer)
    l2n_ref[...] = _rownorm(l2 + d2)
    l3n_ref[...] = _rownorm(l3 + d3)

    @pl.when(i == 0)
    def _():
        acc_ref[...] = jnp.zeros((1, 1), f32)

    acc_ref[...] = acc_ref[...] + jnp.reshape(part, (1, 1))


def _run_main(embk, s1r, s2r, degp, n1, n2, sab, sbb, te, wts):
    rowspec = pl.BlockSpec((RB, EMB), lambda i: (i, 0))
    wspecs = []
    for w in wts:
        wspecs.append(pl.BlockSpec(w.shape, lambda i: (0,) * w.ndim))
    outs = pl.pallas_call(
        _main_kernel,
        grid=(NB,),
        in_specs=[
            rowspec,
            *[rowspec] * 8,
            pl.BlockSpec((RB, NGRAPH), lambda i: (i, 0)),
            rowspec, rowspec,
            pl.BlockSpec((RB, 1), lambda i: (i, 0)),
            pl.BlockSpec((RB, 1), lambda i: (i, 0)),
            pl.BlockSpec((RB, D_EMB), lambda i: (i, 0)),
            *wspecs,
        ],
        out_specs=[
            rowspec, rowspec, rowspec, rowspec,
            pl.BlockSpec((1, 1), lambda i: (0, 0)),
        ],
        out_shape=[
            jax.ShapeDtypeStruct((N, EMB), jnp.float32),
            jax.ShapeDtypeStruct((N, EMB), jnp.float32),
            jax.ShapeDtypeStruct((N, EMB), jnp.float32),
            jax.ShapeDtypeStruct((N, EMB), jnp.float32),
            jax.ShapeDtypeStruct((1, 1), jnp.float32),
        ],
        compiler_params=pltpu.CompilerParams(
            dimension_semantics=("arbitrary",)),
    )(embk, s1r[0], s1r[1], s1r[2], s1r[3], s2r[0], s2r[1], s2r[2], s2r[3],
      degp, n1, n2, sab, sbb, te, *wts)
    return outs


def _infonce_kernel(l4_ref, b2_ref, b3_ref, acc_ref, s2, p2, s3, p3):
    i = pl.program_id(0)
    j = pl.program_id(1)
    a = l4_ref[...] * (1.0 / GCL_TEMP)
    f32 = jnp.float32
    ab = a.astype(jnp.bfloat16)
    dn = (((1,), (1,)), ((), ()))
    log2 = lax.dot_general(ab, b2_ref[...].astype(jnp.bfloat16), dn,
                           preferred_element_type=f32)
    log3 = lax.dot_general(ab, b3_ref[...].astype(jnp.bfloat16), dn,
                           preferred_element_type=f32)
    mfix = 1.0 / GCL_TEMP

    def update(lg, bref, s_ref, p_ref):
        ssum = jnp.sum(jnp.exp(lg - mfix), axis=1, keepdims=True)

        @pl.when(j == 0)
        def _():
            s_ref[...] = ssum

        @pl.when(j > 0)
        def _():
            s_ref[...] = s_ref[...] + ssum

        @pl.when(j == i)
        def _():
            p_ref[...] = jnp.sum(a * bref[...], axis=1, keepdims=True)

    update(log2, b2_ref, s2, p2)
    update(log3, b3_ref, s3, p3)

    @pl.when((i == 0) & (j == 0))
    def _():
        acc_ref[...] = jnp.zeros((1, 1), f32)

    @pl.when(j == NB - 1)
    def _():
        part = (jnp.sum(mfix + jnp.log(s2[...]) - p2[...])
                + jnp.sum(mfix + jnp.log(s3[...]) - p3[...]))
        acc_ref[...] = acc_ref[...] + jnp.reshape(part, (1, 1))


def _run_infonce(l4, l2n, l3n):
    col = pl.BlockSpec((RB, 1), None)
    acc = pl.pallas_call(
        _infonce_kernel,
        grid=(NB, NB),
        in_specs=[
            pl.BlockSpec((RB, EMB), lambda i, j: (i, 0)),
            pl.BlockSpec((RB, EMB), lambda i, j: (j, 0)),
            pl.BlockSpec((RB, EMB), lambda i, j: (j, 0)),
        ],
        out_specs=pl.BlockSpec((1, 1), lambda i, j: (0, 0)),
        out_shape=jax.ShapeDtypeStruct((1, 1), jnp.float32),
        scratch_shapes=[pltpu.VMEM((RB, 1), jnp.float32) for _ in range(4)],
        compiler_params=pltpu.CompilerParams(
            dimension_semantics=("arbitrary", "arbitrary")),
    )(l4, l2n, l3n)
    return acc


def _bpr_kernel(g_ref, out_ref):
    u = g_ref[0]
    p = g_ref[1]
    ng = g_ref[2]
    ue = g_ref[3]
    pe = g_ref[4]
    ne = g_ref[5]
    pos = jnp.sum(u * p, axis=-1)
    neg = jnp.sum(u * ng, axis=-1)
    x = pos - neg
    logsig = jnp.minimum(x, 0.0) - jnp.log1p(jnp.exp(-jnp.abs(x)))
    cf = -jnp.sum(logsig) * (1.0 / B)
    reg = 0.5 * (jnp.sum(ue * ue) + jnp.sum(pe * pe) + jnp.sum(ne * ne)) / B
    out_ref[...] = jnp.reshape(CF_WEIGHT * cf + WEIGHT_DECAY * reg, (1, 1))


def _run_bpr(grows):
    return pl.pallas_call(
        _bpr_kernel,
        in_specs=[pl.BlockSpec((6, B, EMB), lambda: (0, 0, 0))],
        out_specs=pl.BlockSpec((1, 1), lambda: (0, 0)),
        out_shape=jax.ShapeDtypeStruct((1, 1), jnp.float32),
    )(grows)


def _timestep_embedding_const():
    ts = jnp.arange(N, dtype=jnp.float32) % STEPS
    half = D_EMB // 2
    freqs = jnp.exp(-math.log(10000.0)
                    * jnp.arange(half, dtype=jnp.float32) / half)
    a = ts[:, None] * freqs[None, :]
    return jnp.concatenate([jnp.cos(a), jnp.sin(a)], axis=-1)


def kernel(user_idx, pos_item, neg_item, edge_index, ig_edge_index,
           kg_edge_index, cf_edge_index, emb, i_Wt, i_bt, i_W1, i_b1, i_W2,
           i_b2, c_Wt, c_bt, c_W1, c_b1, c_W2, c_b2):
    i32 = jnp.int32
    f32 = jnp.float32

    eis = [edge_index, ig_edge_index, kg_edge_index, cf_edge_index]
    pad_n = EPAD - E
    pad_src = (jnp.arange(pad_n, dtype=i32) * 37) % N
    pad_dst = N + (jnp.arange(pad_n, dtype=i32) % (NP - N))
    srcs = jnp.stack([ei[0].astype(i32) for ei in eis])
    dsts = jnp.stack([ei[1].astype(i32) for ei in eis])
    srcs = jnp.concatenate(
        [srcs, jnp.broadcast_to(pad_src, (NGRAPH, pad_n))], axis=1)
    dsts = jnp.concatenate(
        [dsts, jnp.broadcast_to(pad_dst, (NGRAPH, pad_n))], axis=1)
    goff = (jnp.arange(NGRAPH, dtype=i32) * NP)[:, None]
    dsts_flat = dsts.reshape(-1)
    arr_s = (srcs + goff).reshape(NGRAPH, NS, CHUNKS, CH)
    arr_d = dsts.reshape(NGRAPH, NS, CHUNKS, CH)
    idxcat = jnp.stack([arr_s, arr_d], axis=3).reshape(-1, CH)

    zeros1 = jnp.zeros((NP,), f32)
    zeros2 = jnp.zeros((NP, EMB), f32)

    deg_flat = _sc_degree(dsts_flat, zeros1)
    degp = deg_flat.reshape(NGRAPH, NP).T

    embp = jnp.concatenate([emb.astype(f32),
                            jnp.zeros((NP - N, EMB), f32)], axis=0)
    y1 = _scale_emb(degp, embp)
    s1 = _sc_propagate(idxcat, y1, zeros2)
    y2 = _scale_s1(degp, s1)
    s2 = _sc_propagate(idxcat, y2, zeros2)

    s1r = s1.reshape(NGRAPH, NP, EMB)
    s2r = s2.reshape(NGRAPH, NP, EMB)
    s1g = [s1r[g] for g in range(NGRAPH)]
    s2g = [s2r[g] for g in range(NGRAPH)]

    betas = NOISE_SCALE * jnp.linspace(NOISE_MIN, NOISE_MAX, STEPS)
    ab = jnp.cumprod(1.0 - betas)
    ts = jnp.arange(N) % STEPS
    abt = ab[ts][:, None].astype(f32)
    sab = jnp.sqrt(abt)
    sbb = jnp.sqrt(1.0 - abt)
    n1 = jax.random.normal(jax.random.key(1), (N, EMB), dtype=f32)
    n2 = jax.random.normal(jax.random.key(2), (N, EMB), dtype=f32)
    te = _timestep_embedding_const()

    wts = [i_Wt, i_bt.reshape(1, D_EMB), i_W1[:EMB], i_W1[EMB:],
           i_b1.reshape(1, EMB), i_W2, i_b2.reshape(1, EMB),
           c_Wt, c_bt.reshape(1, D_EMB), c_W1[:EMB], c_W1[EMB:],
           c_b1.reshape(1, EMB), c_W2, c_b2.reshape(1, EMB)]

    all_layer, l4, l2n, l3n, diff_acc = _run_main(
        emb.astype(f32), s1g, s2g, degp[:N], n1, n2, sab, sbb, te, wts)

    nce_acc = _run_infonce(l4, l2n, l3n)

    tables = jnp.concatenate([all_layer, emb.astype(f32)], axis=0)
    idx_all = jnp.concatenate([
        user_idx.astype(i32), pos_item.astype(i32), neg_item.astype(i32),
        user_idx.astype(i32) + N, pos_item.astype(i32) + N,
        neg_item.astype(i32) + N])
    grows = _sc_gather_rows(idx_all, tables).reshape(6, B, EMB)
    bpr = _run_bpr(grows)

    diff_loss = diff_acc[0, 0] * (1.0 / N)
    gcl = nce_acc[0, 0] * (1.0 / N)
    return bpr[0, 0] + diff_loss + GCL_WEIGHT * gcl


# speedup vs baseline: 16.1136x; 1.0093x over previous
import functools
import math

import jax
import jax.numpy as jnp
from jax import lax
from jax.experimental import pallas as pl
from jax.experimental.pallas import tpu as pltpu
from jax.experimental.pallas import tpu_sc as plsc

N = 10000
EMB = 128
E = 320000
NGRAPH = 4
LAYERS = 2
STEPS = 5
NOISE_SCALE = 0.1
NOISE_MIN = 0.0001
NOISE_MAX = 0.02
CF_WEIGHT = 1.0
WEIGHT_DECAY = 0.0001
GCL_WEIGHT = 0.1
GCL_TEMP = 0.2
D_EMB = 10
B = 4096

NC = 2
NS = 16
CH = 128
NBUF = 4
NBUF_P = 2
CHUNKS = 160
EPT = CHUNKS * CH
EPAD = NS * EPT
NP = 10240
RPT = NP // NS

RB = 1000
NB = N // RB


_sc_mesh = plsc.VectorSubcoreMesh(core_axis_name="c", subcore_axis_name="s")


@functools.partial(
    pl.kernel,
    out_type=jax.ShapeDtypeStruct((NGRAPH * NP,), jnp.float32),
    mesh=_sc_mesh,
    scratch_types=[
        pltpu.VMEM_SHARED((NP,), jnp.float32),
        pltpu.VMEM_SHARED((NP,), jnp.float32),
        pltpu.VMEM((CH,), jnp.float32),
        *[pltpu.VMEM((CH,), jnp.int32) for _ in range(NBUF)],
        *[pltpu.SemaphoreType.DMA for _ in range(NBUF)],
    ],
)
def _sc_degree(dsts, zeros1, deg_out, acc0, acc1, ones_v, *rest):
    idx = rest[:NBUF]
    sem = rest[NBUF:]
    cid = lax.axis_index("c")
    sid = lax.axis_index("s")
    accs = [acc0, acc1]
    for j in range(CH // 16):
        ones_v[pl.ds(j * 16, 16)] = jnp.ones((16,), jnp.float32)
    for gl in range(2):
        pltpu.sync_copy(zeros1.at[pl.ds(sid * RPT, RPT)],
                        accs[gl].at[pl.ds(sid * RPT, RPT)])
    plsc.subcore_barrier()
    for gl in range(2):
        g = cid * 2 + gl
        ebase = g * EPAD + sid * EPT

        def body(it, _, gl=gl, ebase=ebase):
            cps = []
            for b in range(NBUF):
                off = pl.multiple_of(ebase + (it * NBUF + b) * CH, CH)
                cps.append(pltpu.async_copy(
                    dsts.at[pl.ds(off, CH)], idx[b], sem[b]))
            for b in range(NBUF):
                cps[b].wait()
                pltpu.sync_copy(ones_v, accs[gl].at[idx[b]], add=True)
            return 0

        lax.fori_loop(0, CHUNKS // NBUF, body, 0)
    plsc.subcore_barrier()
    for gl in range(2):
        g = cid * 2 + gl
        off = pl.multiple_of(g * NP + sid * RPT, 8)
        pltpu.sync_copy(accs[gl].at[pl.ds(sid * RPT, RPT)],
                        deg_out.at[pl.ds(off, RPT)])


@functools.partial(
    pl.kernel,
    out_type=jax.ShapeDtypeStruct((NGRAPH * NP, EMB), jnp.float32),
    mesh=_sc_mesh,
    scratch_types=[
        pltpu.VMEM_SHARED((NP, EMB), jnp.float32),
        pltpu.VMEM((8, CH), jnp.int32),
        pltpu.VMEM((8, CH), jnp.int32),
        *[pltpu.VMEM((CH, EMB), jnp.float32) for _ in range(2)],
        *[pltpu.SemaphoreType.DMA for _ in range(6)],
    ],
)
def _sc_propagate(idxcat, table, zeros2, out, acc, bx, by, r0, r1,
                  smx, smy, sg0, sg1, ss0, ss1):
    cid = lax.axis_index("c")
    sid = lax.axis_index("s")
    rows = [r0, r1]
    sem_g = [sg0, sg1]
    sem_s = [ss0, ss1]
    nbody = CHUNKS // 8

    def fire_idx(buf, sem, g, body_ix, half):
        base = (g * NS + sid) * (2 * CHUNKS)
        off = base + body_ix * 16 + half * 8
        return pltpu.async_copy(idxcat.at[pl.ds(off, 8)], buf, sem)

    def fire_g(ib, j, b):
        return pltpu.async_copy(table.at[ib.at[2 * j]], rows[b], sem_g[b])

    def fire_s(ib, j, b):
        return pltpu.async_copy(rows[b], acc.at[ib.at[2 * j + 1]],
                                sem_s[b], add=True)

    def wait(sem, ref):
        if ref is bx or ref is by:
            dummy = idxcat.at[pl.ds(0, 8)]
        else:
            dummy = table.at[pl.ds(0, CH)]
        pltpu.make_async_copy(dummy, ref, sem).wait()

    for gl in range(2):
        g = cid * 2 + gl
        pltpu.sync_copy(zeros2.at[pl.ds(sid * RPT, RPT)],
                        acc.at[pl.ds(sid * RPT, RPT)])
        plsc.subcore_barrier()

        fire_idx(bx, smx, g, 0, 0).wait()
        fire_idx(by, smy, g, 0, 1)
        fire_g(bx, 0, 0)
        fire_g(bx, 1, 1)

        def body(k, _, g=g):
            last = k == nbody - 1
            wait(sem_g[0], rows[0]); fire_s(bx, 0, 0)
            wait(sem_g[1], rows[1]); fire_s(bx, 1, 1)
            wait(sem_s[0], rows[0]); fire_g(bx, 2, 0)
            wait(sem_s[1], rows[1]); fire_g(bx, 3, 1)
            wait(smy, by)
            wait(sem_g[0], rows[0]); fire_s(bx, 2, 0)
            wait(sem_g[1], rows[1]); fire_s(bx, 3, 1)
            wait(sem_s[0], rows[0]); fire_g(by, 0, 0)
            wait(sem_s[1], rows[1]); fire_g(by, 1, 1)

            @pl.when(jnp.logical_not(last))
            def _():
                fire_idx(bx, smx, g, k + 1, 0)

            wait(sem_g[0], rows[0]); fire_s(by, 0, 0)
            wait(sem_g[1], rows[1]); fire_s(by, 1, 1)
            wait(sem_s[0], rows[0]); fire_g(by, 2, 0)
            wait(sem_s[1], rows[1]); fire_g(by, 3, 1)
            wait(sem_g[0], rows[0]); fire_s(by, 2, 0)
            wait(sem_g[1], rows[1]); fire_s(by, 3, 1)
            wait(sem_s[0], rows[0])

            @pl.when(jnp.logical_not(last))
            def _():
                wait(smx, bx)
                fire_g(bx, 0, 0)

            wait(sem_s[1], rows[1])

            @pl.when(jnp.logical_not(last))
            def _():
                fire_g(bx, 1, 1)
                fire_idx(by, smy, g, k + 1, 1)

            return 0

        lax.fori_loop(0, nbody, body, 0)
        plsc.subcore_barrier()
        off = g * NP + sid * RPT
        pltpu.sync_copy(acc.at[pl.ds(sid * RPT, RPT)],
                        out.at[pl.ds(off, RPT)])
        plsc.subcore_barrier()


NIDX = 6 * B
GPW = NIDX // (NC * NS)
GCH = GPW // CH


@functools.partial(
    pl.kernel,
    out_type=jax.ShapeDtypeStruct((NIDX, EMB), jnp.float32),
    mesh=_sc_mesh,
    scratch_types=[
        *[pltpu.VMEM((CH,), jnp.int32) for _ in range(2)],
        *[pltpu.VMEM((CH, EMB), jnp.float32) for _ in range(2)],
        *[pltpu.SemaphoreType.DMA for _ in range(4)],
    ],
)
def _sc_gather_rows(idx_all, tables, out, i0, i1, r0, r1, si0, si1, sg0, sg1):
    cid = lax.axis_index("c")
    sid = lax.axis_index("s")
    wid = sid * NC + cid
    base = wid * GPW
    idx = [i0, i1]
    rows = [r0, r1]
    sem_i = [si0, si1]
    sem_g = [sg0, sg1]
    for k in range(GCH):
        b = k % 2
        off = pl.multiple_of(base + k * CH, CH)
        pltpu.async_copy(idx_all.at[pl.ds(off, CH)], idx[b], sem_i[b]).wait()
        pltpu.async_copy(tables.at[idx[b]], rows[b], sem_g[b]).wait()
        pltpu.sync_copy(rows[b], out.at[pl.ds(off, CH)])


def _dinv_of(degcol):
    return jnp.where(degcol > 0.0, lax.rsqrt(jnp.maximum(degcol, 1e-30)), 0.0)


def _scale_table_kernel(degp_ref, x_ref, out_ref, *, power):
    g = pl.program_id(0)
    degs = degp_ref[...]
    col = jnp.zeros_like(degs[:, 0:1])
    for k in range(NGRAPH):
        col = col + jnp.where(g == k, degs[:, k:k + 1], 0.0)
    d = _dinv_of(col)
    w = d * d if power == 2 else d
    out_ref[...] = w * x_ref[...]


def _scale_emb(degp, embp):
    out = pl.pallas_call(
        functools.partial(_scale_table_kernel, power=1),
        grid=(NGRAPH, NP // 1024),
        in_specs=[
            pl.BlockSpec((1024, NGRAPH), lambda g, i: (i, 0)),
            pl.BlockSpec((1024, EMB), lambda g, i: (i, 0)),
        ],
        out_specs=pl.BlockSpec((1024, EMB), lambda g, i: (g * (NP // 1024) + i, 0)),
        out_shape=jax.ShapeDtypeStruct((NGRAPH * NP, EMB), jnp.float32),
    )(degp, embp)
    return out


def _scale_s1(degp, s1):
    out = pl.pallas_call(
        functools.partial(_scale_table_kernel, power=2),
        grid=(NGRAPH, NP // 1024),
        in_specs=[
            pl.BlockSpec((1024, NGRAPH), lambda g, i: (i, 0)),
            pl.BlockSpec((1024, EMB), lambda g, i: (g * (NP // 1024) + i, 0)),
        ],
        out_specs=pl.BlockSpec((1024, EMB), lambda g, i: (g * (NP // 1024) + i, 0)),
        out_shape=jax.ShapeDtypeStruct((NGRAPH * NP, EMB), jnp.float32),
    )(degp, s1)
    return out


def _rownorm(x):
    n = jnp.sqrt(jnp.sum(x * x, axis=-1, keepdims=True))
    return x / jnp.maximum(n, 1e-12)


def _main_kernel(emb_ref, s1_ref, s2_ref,
                 degp_ref, n1_ref, n2_ref, sab_ref, sbb_ref, te_ref,
                 iWt, ibt, iW1a, iW1b, ib1, iW2, ib2,
                 cWt, cbt, cW1a, cW1b, cb1, cW2, cb2,
                 all_ref, l4_ref, l2n_ref, l3n_ref, acc_ref):
    i = pl.program_id(0)
    deg = degp_ref[...]
    emb = emb_ref[...]

    def allg(g):
        d = _dinv_of(deg[:, g:g + 1])
        return (emb + d * (s1_ref[g] + s2_ref[g])) * (1.0 / 3.0)

    all_layer = allg(0)
    l2 = allg(1)
    l3 = allg(2)
    l1n = _rownorm(allg(3))

    f32 = jnp.float32

    def mlp(x_t, Wt, bt, W1a, W1b, b1, W2, b2):
        temb = jnp.dot(te_ref[...], Wt[...], preferred_element_type=f32) + bt[...]
        h = jnp.tanh(jnp.dot(x_t, W1a[...], preferred_element_type=f32)
                     + jnp.dot(temb, W1b[...], preferred_element_type=f32)
                     + b1[...])
        return jnp.dot(h, W2[...], preferred_element_type=f32) + b2[...]

    sab = sab_ref[...]
    sbb = sbb_ref[...]
    x_t2 = sab * l2 + sbb * n1_ref[...]
    x_t3 = sab * l3 + sbb * n2_ref[...]
    d2 = mlp(x_t2, iWt, ibt, iW1a, iW1b, ib1, iW2, ib2)
    d3 = mlp(x_t3, cWt, cbt, cW1a, cW1b, cb1, cW2, cb2)

    part = (jnp.sum((l1n - d2) ** 2) + jnp.sum((l1n - d3) ** 2)) * (1.0 / EMB)

    all_ref[...] = all_layer
    l4_ref[...] = _rownorm(all_layer)
    l2n_ref[...] = _rownorm(l2 + d2)
    l3n_ref[...] = _rownorm(l3 + d3)

    @pl.when(i == 0)
    def _():
        acc_ref[...] = jnp.zeros((1, 1), f32)

    acc_ref[...] = acc_ref[...] + jnp.reshape(part, (1, 1))


def _run_main(embk, s1r, s2r, degp, n1, n2, sab, sbb, te, wts):
    rowspec = pl.BlockSpec((RB, EMB), lambda i: (i, 0))
    gspec = pl.BlockSpec((NGRAPH, RB, EMB), lambda i: (0, i, 0))
    wspecs = []
    for w in wts:
        wspecs.append(pl.BlockSpec(w.shape, lambda i: (0,) * w.ndim))
    outs = pl.pallas_call(
        _main_kernel,
        grid=(NB,),
        in_specs=[
            rowspec,
            gspec, gspec,
            pl.BlockSpec((RB, NGRAPH), lambda i: (i, 0)),
            rowspec, rowspec,
            pl.BlockSpec((RB, 1), lambda i: (i, 0)),
            pl.BlockSpec((RB, 1), lambda i: (i, 0)),
            pl.BlockSpec((RB, D_EMB), lambda i: (i, 0)),
            *wspecs,
        ],
        out_specs=[
            rowspec, rowspec, rowspec, rowspec,
            pl.BlockSpec((1, 1), lambda i: (0, 0)),
        ],
        out_shape=[
            jax.ShapeDtypeStruct((N, EMB), jnp.float32),
            jax.ShapeDtypeStruct((N, EMB), jnp.float32),
            jax.ShapeDtypeStruct((N, EMB), jnp.float32),
            jax.ShapeDtypeStruct((N, EMB), jnp.float32),
            jax.ShapeDtypeStruct((1, 1), jnp.float32),
        ],
        compiler_params=pltpu.CompilerParams(
            dimension_semantics=("arbitrary",)),
    )(embk, s1r, s2r, degp, n1, n2, sab, sbb, te, *wts)
    return outs


def _infonce_kernel(l4_ref, b2_ref, b3_ref, acc_ref, s2, p2, s3, p3):
    i = pl.program_id(0)
    j = pl.program_id(1)
    a = l4_ref[...] * (1.0 / GCL_TEMP)
    f32 = jnp.float32
    ab = a.astype(jnp.bfloat16)
    dn = (((1,), (1,)), ((), ()))
    log2 = lax.dot_general(ab, b2_ref[...].astype(jnp.bfloat16), dn,
                           preferred_element_type=f32)
    log3 = lax.dot_general(ab, b3_ref[...].astype(jnp.bfloat16), dn,
                           preferred_element_type=f32)
    def update(lg, bref, s_ref, p_ref):
        ssum = jnp.sum(jnp.exp(lg), axis=1, keepdims=True)

        @pl.when(j == 0)
        def _():
            s_ref[...] = ssum

        @pl.when(j > 0)
        def _():
            s_ref[...] = s_ref[...] + ssum

        @pl.when(j == i)
        def _():
            p_ref[...] = jnp.sum(a * bref[...], axis=1, keepdims=True)

    update(log2, b2_ref, s2, p2)
    update(log3, b3_ref, s3, p3)

    @pl.when((i == 0) & (j == 0))
    def _():
        acc_ref[...] = jnp.zeros((1, 1), f32)

    @pl.when(j == NB - 1)
    def _():
        part = (jnp.sum(jnp.log(s2[...]) - p2[...])
                + jnp.sum(jnp.log(s3[...]) - p3[...]))
        acc_ref[...] = acc_ref[...] + jnp.reshape(part, (1, 1))


def _run_infonce(l4, l2n, l3n):
    col = pl.BlockSpec((RB, 1), None)
    acc = pl.pallas_call(
        _infonce_kernel,
        grid=(NB, NB),
        in_specs=[
            pl.BlockSpec((RB, EMB), lambda i, j: (i, 0)),
            pl.BlockSpec((RB, EMB), lambda i, j: (j, 0)),
            pl.BlockSpec((RB, EMB), lambda i, j: (j, 0)),
        ],
        out_specs=pl.BlockSpec((1, 1), lambda i, j: (0, 0)),
        out_shape=jax.ShapeDtypeStruct((1, 1), jnp.float32),
        scratch_shapes=[pltpu.VMEM((RB, 1), jnp.float32) for _ in range(4)],
        compiler_params=pltpu.CompilerParams(
            dimension_semantics=("arbitrary", "arbitrary")),
    )(l4, l2n, l3n)
    return acc


def _bpr_kernel(g_ref, out_ref):
    u = g_ref[0]
    p = g_ref[1]
    ng = g_ref[2]
    ue = g_ref[3]
    pe = g_ref[4]
    ne = g_ref[5]
    pos = jnp.sum(u * p, axis=-1)
    neg = jnp.sum(u * ng, axis=-1)
    x = pos - neg
    logsig = jnp.minimum(x, 0.0) - jnp.log1p(jnp.exp(-jnp.abs(x)))
    cf = -jnp.sum(logsig) * (1.0 / B)
    reg = 0.5 * (jnp.sum(ue * ue) + jnp.sum(pe * pe) + jnp.sum(ne * ne)) / B
    out_ref[...] = jnp.reshape(CF_WEIGHT * cf + WEIGHT_DECAY * reg, (1, 1))


def _run_bpr(grows):
    return pl.pallas_call(
        _bpr_kernel,
        in_specs=[pl.BlockSpec((6, B, EMB), lambda: (0, 0, 0))],
        out_specs=pl.BlockSpec((1, 1), lambda: (0, 0)),
        out_shape=jax.ShapeDtypeStruct((1, 1), jnp.float32),
    )(grows)


def _timestep_embedding_const():
    ts = jnp.arange(N, dtype=jnp.float32) % STEPS
    half = D_EMB // 2
    freqs = jnp.exp(-math.log(10000.0)
                    * jnp.arange(half, dtype=jnp.float32) / half)
    a = ts[:, None] * freqs[None, :]
    return jnp.concatenate([jnp.cos(a), jnp.sin(a)], axis=-1)


def kernel(user_idx, pos_item, neg_item, edge_index, ig_edge_index,
           kg_edge_index, cf_edge_index, emb, i_Wt, i_bt, i_W1, i_b1, i_W2,
           i_b2, c_Wt, c_bt, c_W1, c_b1, c_W2, c_b2):
    i32 = jnp.int32
    f32 = jnp.float32

    eis = [edge_index, ig_edge_index, kg_edge_index, cf_edge_index]
    pad_n = EPAD - E
    pad_src = (jnp.arange(pad_n, dtype=i32) * 37) % N
    pad_dst = N + (jnp.arange(pad_n, dtype=i32) % (NP - N))
    srcs = jnp.stack([ei[0].astype(i32) for ei in eis])
    dsts = jnp.stack([ei[1].astype(i32) for ei in eis])
    srcs = jnp.concatenate(
        [srcs, jnp.broadcast_to(pad_src, (NGRAPH, pad_n))], axis=1)
    dsts = jnp.concatenate(
        [dsts, jnp.broadcast_to(pad_dst, (NGRAPH, pad_n))], axis=1)
    goff = (jnp.arange(NGRAPH, dtype=i32) * NP)[:, None]
    dsts_flat = dsts.reshape(-1)
    arr_s = (srcs + goff).reshape(NGRAPH, NS, CHUNKS, CH)
    arr_d = dsts.reshape(NGRAPH, NS, CHUNKS, CH)
    idxcat = jnp.stack([arr_s, arr_d], axis=3).reshape(-1, CH)

    zeros1 = jnp.zeros((NP,), f32)
    zeros2 = jnp.zeros((NP, EMB), f32)

    deg_flat = _sc_degree(dsts_flat, zeros1)
    degp = deg_flat.reshape(NGRAPH, NP).T

    embp = jnp.concatenate([emb.astype(f32),
                            jnp.zeros((NP - N, EMB), f32)], axis=0)
    y1 = _scale_emb(degp, embp)
    s1 = _sc_propagate(idxcat, y1, zeros2)
    y2 = _scale_s1(degp, s1)
    s2 = _sc_propagate(idxcat, y2, zeros2)

    s1r = s1.reshape(NGRAPH, NP, EMB)
    s2r = s2.reshape(NGRAPH, NP, EMB)

    betas = NOISE_SCALE * jnp.linspace(NOISE_MIN, NOISE_MAX, STEPS)
    ab = jnp.cumprod(1.0 - betas)
    ts = jnp.arange(N) % STEPS
    abt = ab[ts][:, None].astype(f32)
    sab = jnp.sqrt(abt)
    sbb = jnp.sqrt(1.0 - abt)
    n1 = jax.random.normal(jax.random.key(1), (N, EMB), dtype=f32)
    n2 = jax.random.normal(jax.random.key(2), (N, EMB), dtype=f32)
    te = _timestep_embedding_const()

    wts = [i_Wt, i_bt.reshape(1, D_EMB), i_W1[:EMB], i_W1[EMB:],
           i_b1.reshape(1, EMB), i_W2, i_b2.reshape(1, EMB),
           c_Wt, c_bt.reshape(1, D_EMB), c_W1[:EMB], c_W1[EMB:],
           c_b1.reshape(1, EMB), c_W2, c_b2.reshape(1, EMB)]

    all_layer, l4, l2n, l3n, diff_acc = _run_main(
        emb.astype(f32), s1r, s2r, degp[:N], n1, n2, sab, sbb, te, wts)

    nce_acc = _run_infonce(l4, l2n, l3n)

    tables = jnp.concatenate([all_layer, emb.astype(f32)], axis=0)
    idx_all = jnp.concatenate([
        user_idx.astype(i32), pos_item.astype(i32), neg_item.astype(i32),
        user_idx.astype(i32) + N, pos_item.astype(i32) + N,
        neg_item.astype(i32) + N])
    grows = _sc_gather_rows(idx_all, tables).reshape(6, B, EMB)
    bpr = _run_bpr(grows)

    diff_loss = diff_acc[0, 0] * (1.0 / N)
    gcl = nce_acc[0, 0] * (1.0 / N)
    return bpr[0, 0] + diff_loss + GCL_WEIGHT * gcl


# speedup vs baseline: 16.8758x; 1.0473x over previous
import functools
import math

import jax
import jax.numpy as jnp
from jax import lax
from jax.experimental import pallas as pl
from jax.experimental.pallas import tpu as pltpu
from jax.experimental.pallas import tpu_sc as plsc

N = 10000
EMB = 128
E = 320000
NGRAPH = 4
LAYERS = 2
STEPS = 5
NOISE_SCALE = 0.1
NOISE_MIN = 0.0001
NOISE_MAX = 0.02
CF_WEIGHT = 1.0
WEIGHT_DECAY = 0.0001
GCL_WEIGHT = 0.1
GCL_TEMP = 0.2
D_EMB = 10
B = 4096

NC = 2
NS = 16
CH = 128
NBUF = 8
NBUF_P = 2
CHUNKS = 160
EPT = CHUNKS * CH
EPAD = NS * EPT
NP = 10240
RPT = NP // NS

RB = 1000
NB = N // RB


_sc_mesh = plsc.VectorSubcoreMesh(core_axis_name="c", subcore_axis_name="s")


@functools.partial(
    pl.kernel,
    out_type=jax.ShapeDtypeStruct((NGRAPH * NP,), jnp.float32),
    mesh=_sc_mesh,
    scratch_types=[
        pltpu.VMEM_SHARED((NP,), jnp.float32),
        pltpu.VMEM_SHARED((NP,), jnp.float32),
        pltpu.VMEM((CH,), jnp.float32),
        *[pltpu.VMEM((CH,), jnp.int32) for _ in range(NBUF)],
        *[pltpu.SemaphoreType.DMA for _ in range(2 * NBUF)],
    ],
)
def _sc_degree(dsts, zeros1, deg_out, acc0, acc1, ones_v, *rest):
    idx = rest[:NBUF]
    sem_i = rest[NBUF:2 * NBUF]
    sem_c = rest[2 * NBUF:]
    cid = lax.axis_index("c")
    sid = lax.axis_index("s")
    accs = [acc0, acc1]
    for j in range(CH // 16):
        ones_v[pl.ds(j * 16, 16)] = jnp.ones((16,), jnp.float32)
    for gl in range(2):
        pltpu.sync_copy(zeros1.at[pl.ds(sid * RPT, RPT)],
                        accs[gl].at[pl.ds(sid * RPT, RPT)])
    plsc.subcore_barrier()
    for gl in range(2):
        g = cid * 2 + gl
        ebase = g * EPAD + sid * EPT
        nbody = CHUNKS // NBUF

        def fire_i(b, it, ebase=ebase):
            off = pl.multiple_of(ebase + (it * NBUF + b) * CH, CH)
            return pltpu.async_copy(dsts.at[pl.ds(off, CH)], idx[b], sem_i[b])

        for b in range(NBUF):
            fire_i(b, 0)

        def body(it, _, gl=gl, nbody=nbody):
            for b in range(NBUF):
                pltpu.make_async_copy(dsts.at[pl.ds(0, CH)],
                                      idx[b], sem_i[b]).wait()
                pltpu.async_copy(ones_v, accs[gl].at[idx[b]], sem_c[b],
                                 add=True)
            for b in range(NBUF):
                pltpu.make_async_copy(zeros1.at[pl.ds(0, CH)],
                                      ones_v, sem_c[b]).wait()

                @pl.when(it < nbody - 1)
                def _(b=b, it=it):
                    fire_i(b, it + 1)

            return 0

        lax.fori_loop(0, nbody, body, 0)
    plsc.subcore_barrier()
    for gl in range(2):
        g = cid * 2 + gl
        off = pl.multiple_of(g * NP + sid * RPT, 8)
        pltpu.sync_copy(accs[gl].at[pl.ds(sid * RPT, RPT)],
                        deg_out.at[pl.ds(off, RPT)])


@functools.partial(
    pl.kernel,
    out_type=jax.ShapeDtypeStruct((NGRAPH * NP, EMB), jnp.float32),
    mesh=_sc_mesh,
    scratch_types=[
        pltpu.VMEM_SHARED((NP, EMB), jnp.float32),
        pltpu.VMEM((8, CH), jnp.int32),
        pltpu.VMEM((8, CH), jnp.int32),
        *[pltpu.VMEM((CH, EMB), jnp.float32) for _ in range(2)],
        *[pltpu.SemaphoreType.DMA for _ in range(6)],
    ],
)
def _sc_propagate(idxcat, table, zeros2, out, acc, bx, by, r0, r1,
                  smx, smy, sg0, sg1, ss0, ss1):
    cid = lax.axis_index("c")
    sid = lax.axis_index("s")
    rows = [r0, r1]
    sem_g = [sg0, sg1]
    sem_s = [ss0, ss1]
    nbody = CHUNKS // 8

    def fire_idx(buf, sem, g, body_ix, half):
        base = (g * NS + sid) * (2 * CHUNKS)
        off = base + body_ix * 16 + half * 8
        return pltpu.async_copy(idxcat.at[pl.ds(off, 8)], buf, sem)

    def fire_g(ib, j, b):
        return pltpu.async_copy(table.at[ib.at[2 * j]], rows[b], sem_g[b])

    def fire_s(ib, j, b):
        return pltpu.async_copy(rows[b], acc.at[ib.at[2 * j + 1]],
                                sem_s[b], add=True)

    def wait(sem, ref):
        if ref is bx or ref is by:
            dummy = idxcat.at[pl.ds(0, 8)]
        else:
            dummy = table.at[pl.ds(0, CH)]
        pltpu.make_async_copy(dummy, ref, sem).wait()

    for gl in range(2):
        g = cid * 2 + gl
        pltpu.sync_copy(zeros2.at[pl.ds(sid * RPT, RPT)],
                        acc.at[pl.ds(sid * RPT, RPT)])
        plsc.subcore_barrier()

        fire_idx(bx, smx, g, 0, 0).wait()
        fire_idx(by, smy, g, 0, 1)
        fire_g(bx, 0, 0)
        fire_g(bx, 1, 1)

        def body(k, _, g=g):
            last = k == nbody - 1
            wait(sem_g[0], rows[0]); fire_s(bx, 0, 0)
            wait(sem_g[1], rows[1]); fire_s(bx, 1, 1)
            wait(sem_s[0], rows[0]); fire_g(bx, 2, 0)
            wait(sem_s[1], rows[1]); fire_g(bx, 3, 1)
            wait(smy, by)
            wait(sem_g[0], rows[0]); fire_s(bx, 2, 0)
            wait(sem_g[1], rows[1]); fire_s(bx, 3, 1)
            wait(sem_s[0], rows[0]); fire_g(by, 0, 0)
            wait(sem_s[1], rows[1]); fire_g(by, 1, 1)

            @pl.when(jnp.logical_not(last))
            def _():
                fire_idx(bx, smx, g, k + 1, 0)

            wait(sem_g[0], rows[0]); fire_s(by, 0, 0)
            wait(sem_g[1], rows[1]); fire_s(by, 1, 1)
            wait(sem_s[0], rows[0]); fire_g(by, 2, 0)
            wait(sem_s[1], rows[1]); fire_g(by, 3, 1)
            wait(sem_g[0], rows[0]); fire_s(by, 2, 0)
            wait(sem_g[1], rows[1]); fire_s(by, 3, 1)
            wait(sem_s[0], rows[0])

            @pl.when(jnp.logical_not(last))
            def _():
                wait(smx, bx)
                fire_g(bx, 0, 0)

            wait(sem_s[1], rows[1])

            @pl.when(jnp.logical_not(last))
            def _():
                fire_g(bx, 1, 1)
                fire_idx(by, smy, g, k + 1, 1)

            return 0

        lax.fori_loop(0, nbody, body, 0)
        plsc.subcore_barrier()
        off = g * NP + sid * RPT
        pltpu.sync_copy(acc.at[pl.ds(sid * RPT, RPT)],
                        out.at[pl.ds(off, RPT)])
        plsc.subcore_barrier()


NIDX = 6 * B
GPW = NIDX // (NC * NS)
GCH = GPW // CH


@functools.partial(
    pl.kernel,
    out_type=jax.ShapeDtypeStruct((NIDX, EMB), jnp.float32),
    mesh=_sc_mesh,
    scratch_types=[
        *[pltpu.VMEM((CH,), jnp.int32) for _ in range(2)],
        *[pltpu.VMEM((CH, EMB), jnp.float32) for _ in range(2)],
        *[pltpu.SemaphoreType.DMA for _ in range(4)],
    ],
)
def _sc_gather_rows(idx_all, tables, out, i0, i1, r0, r1, si0, si1, sg0, sg1):
    cid = lax.axis_index("c")
    sid = lax.axis_index("s")
    wid = sid * NC + cid
    base = wid * GPW
    idx = [i0, i1]
    rows = [r0, r1]
    sem_i = [si0, si1]
    sem_g = [sg0, sg1]
    for k in range(GCH):
        b = k % 2
        off = pl.multiple_of(base + k * CH, CH)
        pltpu.async_copy(idx_all.at[pl.ds(off, CH)], idx[b], sem_i[b]).wait()
        pltpu.async_copy(tables.at[idx[b]], rows[b], sem_g[b]).wait()
        pltpu.sync_copy(rows[b], out.at[pl.ds(off, CH)])


def _dinv_of(degcol):
    return jnp.where(degcol > 0.0, lax.rsqrt(jnp.maximum(degcol, 1e-30)), 0.0)


def _scale_table_kernel(degp_ref, x_ref, out_ref, *, power):
    g = pl.program_id(0)
    degs = degp_ref[...]
    col = jnp.zeros_like(degs[:, 0:1])
    for k in range(NGRAPH):
        col = col + jnp.where(g == k, degs[:, k:k + 1], 0.0)
    d = _dinv_of(col)
    w = d * d if power == 2 else d
    out_ref[...] = w * x_ref[...]


def _scale_emb(degp, embp):
    out = pl.pallas_call(
        functools.partial(_scale_table_kernel, power=1),
        grid=(NGRAPH, NP // 1024),
        in_specs=[
            pl.BlockSpec((1024, NGRAPH), lambda g, i: (i, 0)),
            pl.BlockSpec((1024, EMB), lambda g, i: (i, 0)),
        ],
        out_specs=pl.BlockSpec((1024, EMB), lambda g, i: (g * (NP // 1024) + i, 0)),
        out_shape=jax.ShapeDtypeStruct((NGRAPH * NP, EMB), jnp.float32),
    )(degp, embp)
    return out


def _scale_s1(degp, s1):
    out = pl.pallas_call(
        functools.partial(_scale_table_kernel, power=2),
        grid=(NGRAPH, NP // 1024),
        in_specs=[
            pl.BlockSpec((1024, NGRAPH), lambda g, i: (i, 0)),
            pl.BlockSpec((1024, EMB), lambda g, i: (g * (NP // 1024) + i, 0)),
        ],
        out_specs=pl.BlockSpec((1024, EMB), lambda g, i: (g * (NP // 1024) + i, 0)),
        out_shape=jax.ShapeDtypeStruct((NGRAPH * NP, EMB), jnp.float32),
    )(degp, s1)
    return out


def _rownorm(x):
    n = jnp.sqrt(jnp.sum(x * x, axis=-1, keepdims=True))
    return x / jnp.maximum(n, 1e-12)


def _main_kernel(emb_ref, s1_ref, s2_ref,
                 degp_ref, n1_ref, n2_ref, sab_ref, sbb_ref, te_ref,
                 iWt, ibt, iW1a, iW1b, ib1, iW2, ib2,
                 cWt, cbt, cW1a, cW1b, cb1, cW2, cb2,
                 all_ref, l4_ref, l2n_ref, l3n_ref, acc_ref):
    i = pl.program_id(0)
    deg = degp_ref[...]
    emb = emb_ref[...]

    def allg(g):
        d = _dinv_of(deg[:, g:g + 1])
        return (emb + d * (s1_ref[g] + s2_ref[g])) * (1.0 / 3.0)

    all_layer = allg(0)
    l2 = allg(1)
    l3 = allg(2)
    l1n = _rownorm(allg(3))

    f32 = jnp.float32

    def mlp(x_t, Wt, bt, W1a, W1b, b1, W2, b2):
        temb = jnp.dot(te_ref[...], Wt[...], preferred_element_type=f32) + bt[...]
        h = jnp.tanh(jnp.dot(x_t, W1a[...], preferred_element_type=f32)
                     + jnp.dot(temb, W1b[...], preferred_element_type=f32)
                     + b1[...])
        return jnp.dot(h, W2[...], preferred_element_type=f32) + b2[...]

    sab = sab_ref[...]
    sbb = sbb_ref[...]
    x_t2 = sab * l2 + sbb * n1_ref[...]
    x_t3 = sab * l3 + sbb * n2_ref[...]
    d2 = mlp(x_t2, iWt, ibt, iW1a, iW1b, ib1, iW2, ib2)
    d3 = mlp(x_t3, cWt, cbt, cW1a, cW1b, cb1, cW2, cb2)

    part = (jnp.sum((l1n - d2) ** 2) + jnp.sum((l1n - d3) ** 2)) * (1.0 / EMB)

    all_ref[...] = all_layer
    l4_ref[...] = _rownorm(all_layer)
    l2n_ref[...] = _rownorm(l2 + d2)
    l3n_ref[...] = _rownorm(l3 + d3)

    @pl.when(i == 0)
    def _():
        acc_ref[...] = jnp.zeros((1, 1), f32)

    acc_ref[...] = acc_ref[...] + jnp.reshape(part, (1, 1))


def _run_main(embk, s1r, s2r, degp, n1, n2, sab, sbb, te, wts):
    rowspec = pl.BlockSpec((RB, EMB), lambda i: (i, 0))
    gspec = pl.BlockSpec((NGRAPH, RB, EMB), lambda i: (0, i, 0))
    wspecs = []
    for w in wts:
        wspecs.append(pl.BlockSpec(w.shape, lambda i: (0,) * w.ndim))
    outs = pl.pallas_call(
        _main_kernel,
        grid=(NB,),
        in_specs=[
            rowspec,
            gspec, gspec,
            pl.BlockSpec((RB, NGRAPH), lambda i: (i, 0)),
            rowspec, rowspec,
            pl.BlockSpec((RB, 1), lambda i: (i, 0)),
            pl.BlockSpec((RB, 1), lambda i: (i, 0)),
            pl.BlockSpec((RB, D_EMB), lambda i: (i, 0)),
            *wspecs,
        ],
        out_specs=[
            rowspec, rowspec, rowspec, rowspec,
            pl.BlockSpec((1, 1), lambda i: (0, 0)),
        ],
        out_shape=[
            jax.ShapeDtypeStruct((N, EMB), jnp.float32),
            jax.ShapeDtypeStruct((N, EMB), jnp.float32),
            jax.ShapeDtypeStruct((N, EMB), jnp.float32),
            jax.ShapeDtypeStruct((N, EMB), jnp.float32),
            jax.ShapeDtypeStruct((1, 1), jnp.float32),
        ],
        compiler_params=pltpu.CompilerParams(
            dimension_semantics=("arbitrary",)),
    )(embk, s1r, s2r, degp, n1, n2, sab, sbb, te, *wts)
    return outs


CB = 2000
NCB = N // CB


def _infonce_kernel(l4_ref, b2_ref, b3_ref, acc_ref, s2, p2, s3, p3):
    i = pl.program_id(0)
    j = pl.program_id(1)
    a = l4_ref[...] * (1.0 / GCL_TEMP)
    f32 = jnp.float32
    ab = a.astype(jnp.bfloat16)
    dn = (((1,), (1,)), ((), ()))
    log2 = lax.dot_general(ab, b2_ref[...].astype(jnp.bfloat16), dn,
                           preferred_element_type=f32)
    log3 = lax.dot_general(ab, b3_ref[...].astype(jnp.bfloat16), dn,
                           preferred_element_type=f32)

    def update(lg, bref, s_ref, p_ref):
        ssum = jnp.sum(jnp.exp(lg), axis=1, keepdims=True)

        @pl.when(j == 0)
        def _():
            s_ref[...] = ssum

        @pl.when(j > 0)
        def _():
            s_ref[...] = s_ref[...] + ssum

        @pl.when((j == i // 2) & (i % 2 == 0))
        def _():
            p_ref[...] = jnp.sum(a * bref[0:RB], axis=1, keepdims=True)

        @pl.when((j == i // 2) & (i % 2 == 1))
        def _():
            p_ref[...] = jnp.sum(a * bref[RB:2 * RB], axis=1, keepdims=True)

    update(log2, b2_ref, s2, p2)
    update(log3, b3_ref, s3, p3)

    @pl.when((i == 0) & (j == 0))
    def _():
        acc_ref[...] = jnp.zeros((1, 1), f32)

    @pl.when(j == NCB - 1)
    def _():
        part = (jnp.sum(jnp.log(s2[...]) - p2[...])
                + jnp.sum(jnp.log(s3[...]) - p3[...]))
        acc_ref[...] = acc_ref[...] + jnp.reshape(part, (1, 1))


def _run_infonce(l4, l2n, l3n):
    acc = pl.pallas_call(
        _infonce_kernel,
        grid=(NB, NCB),
        in_specs=[
            pl.BlockSpec((RB, EMB), lambda i, j: (i, 0)),
            pl.BlockSpec((CB, EMB), lambda i, j: (j, 0)),
            pl.BlockSpec((CB, EMB), lambda i, j: (j, 0)),
        ],
        out_specs=pl.BlockSpec((1, 1), lambda i, j: (0, 0)),
        out_shape=jax.ShapeDtypeStruct((1, 1), jnp.float32),
        scratch_shapes=[pltpu.VMEM((RB, 1), jnp.float32) for _ in range(4)],
        compiler_params=pltpu.CompilerParams(
            dimension_semantics=("arbitrary", "arbitrary")),
    )(l4, l2n, l3n)
    return acc


def _bpr_kernel(g_ref, out_ref):
    u = g_ref[0]
    p = g_ref[1]
    ng = g_ref[2]
    ue = g_ref[3]
    pe = g_ref[4]
    ne = g_ref[5]
    pos = jnp.sum(u * p, axis=-1)
    neg = jnp.sum(u * ng, axis=-1)
    x = pos - neg
    logsig = jnp.minimum(x, 0.0) - jnp.log1p(jnp.exp(-jnp.abs(x)))
    cf = -jnp.sum(logsig) * (1.0 / B)
    reg = 0.5 * (jnp.sum(ue * ue) + jnp.sum(pe * pe) + jnp.sum(ne * ne)) / B
    out_ref[...] = jnp.reshape(CF_WEIGHT * cf + WEIGHT_DECAY * reg, (1, 1))


def _run_bpr(grows):
    return pl.pallas_call(
        _bpr_kernel,
        in_specs=[pl.BlockSpec((6, B, EMB), lambda: (0, 0, 0))],
        out_specs=pl.BlockSpec((1, 1), lambda: (0, 0)),
        out_shape=jax.ShapeDtypeStruct((1, 1), jnp.float32),
    )(grows)


def _timestep_embedding_const():
    ts = jnp.arange(N, dtype=jnp.float32) % STEPS
    half = D_EMB // 2
    freqs = jnp.exp(-math.log(10000.0)
                    * jnp.arange(half, dtype=jnp.float32) / half)
    a = ts[:, None] * freqs[None, :]
    return jnp.concatenate([jnp.cos(a), jnp.sin(a)], axis=-1)


def kernel(user_idx, pos_item, neg_item, edge_index, ig_edge_index,
           kg_edge_index, cf_edge_index, emb, i_Wt, i_bt, i_W1, i_b1, i_W2,
           i_b2, c_Wt, c_bt, c_W1, c_b1, c_W2, c_b2):
    i32 = jnp.int32
    f32 = jnp.float32

    eis = [edge_index, ig_edge_index, kg_edge_index, cf_edge_index]
    pad_n = EPAD - E
    pad_src = (jnp.arange(pad_n, dtype=i32) * 37) % N
    pad_dst = N + (jnp.arange(pad_n, dtype=i32) % (NP - N))
    srcs = jnp.stack([ei[0].astype(i32) for ei in eis])
    dsts = jnp.stack([ei[1].astype(i32) for ei in eis])
    srcs = jnp.concatenate(
        [srcs, jnp.broadcast_to(pad_src, (NGRAPH, pad_n))], axis=1)
    dsts = jnp.concatenate(
        [dsts, jnp.broadcast_to(pad_dst, (NGRAPH, pad_n))], axis=1)
    goff = (jnp.arange(NGRAPH, dtype=i32) * NP)[:, None]
    dsts_flat = dsts.reshape(-1)
    arr_s = (srcs + goff).reshape(NGRAPH, NS, CHUNKS, CH)
    arr_d = dsts.reshape(NGRAPH, NS, CHUNKS, CH)
    idxcat = jnp.stack([arr_s, arr_d], axis=3).reshape(-1, CH)

    zeros1 = jnp.zeros((NP,), f32)
    zeros2 = jnp.zeros((NP, EMB), f32)

    deg_flat = _sc_degree(dsts_flat, zeros1)
    degp = deg_flat.reshape(NGRAPH, NP).T

    embp = jnp.concatenate([emb.astype(f32),
                            jnp.zeros((NP - N, EMB), f32)], axis=0)
    y1 = _scale_emb(degp, embp)
    s1 = _sc_propagate(idxcat, y1, zeros2)
    y2 = _scale_s1(degp, s1)
    s2 = _sc_propagate(idxcat, y2, zeros2)

    s1r = s1.reshape(NGRAPH, NP, EMB)
    s2r = s2.reshape(NGRAPH, NP, EMB)

    betas = NOISE_SCALE * jnp.linspace(NOISE_MIN, NOISE_MAX, STEPS)
    ab = jnp.cumprod(1.0 - betas)
    ts = jnp.arange(N) % STEPS
    abt = ab[ts][:, None].astype(f32)
    sab = jnp.sqrt(abt)
    sbb = jnp.sqrt(1.0 - abt)
    n1 = jax.random.normal(jax.random.key(1), (N, EMB), dtype=f32)
    n2 = jax.random.normal(jax.random.key(2), (N, EMB), dtype=f32)
    te = _timestep_embedding_const()

    wts = [i_Wt, i_bt.reshape(1, D_EMB), i_W1[:EMB], i_W1[EMB:],
           i_b1.reshape(1, EMB), i_W2, i_b2.reshape(1, EMB),
           c_Wt, c_bt.reshape(1, D_EMB), c_W1[:EMB], c_W1[EMB:],
           c_b1.reshape(1, EMB), c_W2, c_b2.reshape(1, EMB)]

    all_layer, l4, l2n, l3n, diff_acc = _run_main(
        emb.astype(f32), s1r, s2r, degp[:N], n1, n2, sab, sbb, te, wts)

    nce_acc = _run_infonce(l4, l2n, l3n)

    tables = jnp.concatenate([all_layer, emb.astype(f32)], axis=0)
    idx_all = jnp.concatenate([
        user_idx.astype(i32), pos_item.astype(i32), neg_item.astype(i32),
        user_idx.astype(i32) + N, pos_item.astype(i32) + N,
        neg_item.astype(i32) + N])
    grows = _sc_gather_rows(idx_all, tables).reshape(6, B, EMB)
    bpr = _run_bpr(grows)

    diff_loss = diff_acc[0, 0] * (1.0 / N)
    gcl = nce_acc[0, 0] * (1.0 / N)
    return bpr[0, 0] + diff_loss + GCL_WEIGHT * gcl
